# baseline (speedup 1.0000x reference)
import functools

import jax
import jax.numpy as jnp
import numpy as np
from jax.experimental import pallas as pl
from jax.experimental.pallas import tpu as pltpu
from jax.experimental.pallas import tpu_sc as plsc

D = 1024
H = 16
DH = 64
FF = 4096
KS = 5
S = 2048
BLK = 256
NBLK = S // BLK
INV_SCALE = 1.0 / np.sqrt(DH)


def _gelu(x):
    return 0.5 * x * (1.0 + jax.lax.erf(x * np.float32(1.0 / np.sqrt(2.0))))


def _ln(x, g, b):
    m = x.mean(-1, keepdims=True)
    v = ((x - m) ** 2).mean(-1, keepdims=True)
    return (x - m) / jnp.sqrt(v + 1e-5) * g + b


def _dot(a, b):
    return jnp.dot(a, b, preferred_element_type=jnp.float32)


def _bdot(a, b):
    return jnp.dot(a.astype(jnp.bfloat16), b,
                   preferred_element_type=jnp.float32)


def _router_conv_kernel(prev_ref, cur_ref, next_ref, rw1_ref, rb1_ref,
                        rw2_ref, rb2_ref, dww_ref, dwb_ref,
                        conv_ref, cls_ref, m10stats_ref):
    i = pl.program_id(0)
    x = cur_ref[...]
    h = _gelu(_dot(x, rw1_ref[...]) + rb1_ref[...])
    lg = _dot(h, rw2_ref[...]) + rb2_ref[...]
    mx = lg.max(axis=-1, keepdims=True)
    e = jnp.exp(lg - mx)
    conf0 = e[:, 0:1] / e.sum(axis=-1, keepdims=True)
    l0, l1, l2 = lg[:, 0:1], lg[:, 1:2], lg[:, 2:3]
    packets = jnp.where(l0 == mx, 0,
                        jnp.where(l1 == mx, 1,
                                  jnp.where(l2 == mx, 2, 3))).astype(jnp.int32)
    m00 = (packets == 0) & (conf0 > 0.99)
    cls = jnp.where(m00, 4, packets)
    cls_ref[0, 0, :] = cls.reshape(1, BLK)[0, :]
    m10 = (packets == 2).astype(jnp.float32)
    contrib = (x * m10).sum(axis=0, keepdims=True)
    ccnt = jnp.full((1, D), m10.sum(), jnp.float32)
    stats = jnp.concatenate([contrib, ccnt], axis=0)

    @pl.when(i == 0)
    def _():
        m10stats_ref[...] = stats

    @pl.when(i > 0)
    def _():
        m10stats_ref[...] += stats

    zero2 = jnp.zeros((2, D), jnp.float32)
    top = jnp.where(i > 0, prev_ref[BLK - 2:BLK, :], zero2)
    bot = jnp.where(i < NBLK - 1, next_ref[0:2, :], zero2)
    ext = jnp.concatenate([top, x, bot], axis=0)
    acc = dwb_ref[...]
    for j in range(KS):
        acc = acc + ext[j:j + BLK, :] * dww_ref[j:j + 1, :]
    conv_ref[...] = _gelu(acc)


def _router_conv(hs2d, p):
    rw1 = p['r_w1']
    rb1 = p['r_b1'].reshape(1, 64)
    rw2 = p['r_w2']
    rb2 = p['r_b2'].reshape(1, 4)
    dww = jnp.transpose(p['dw_w'][:, 0, :], (1, 0))
    dwb = p['dw_b'].reshape(1, D)
    conv, cls3, m10stats = pl.pallas_call(
        _router_conv_kernel,
        grid=(NBLK,),
        in_specs=[
            pl.BlockSpec((BLK, D), lambda i: (jnp.maximum(i - 1, 0), 0)),
            pl.BlockSpec((BLK, D), lambda i: (i, 0)),
            pl.BlockSpec((BLK, D), lambda i: (jnp.minimum(i + 1, NBLK - 1), 0)),
            pl.BlockSpec((D, 64), lambda i: (0, 0)),
            pl.BlockSpec((1, 64), lambda i: (0, 0)),
            pl.BlockSpec((64, 4), lambda i: (0, 0)),
            pl.BlockSpec((1, 4), lambda i: (0, 0)),
            pl.BlockSpec((KS, D), lambda i: (0, 0)),
            pl.BlockSpec((1, D), lambda i: (0, 0)),
        ],
        out_specs=[
            pl.BlockSpec((BLK, D), lambda i: (i, 0)),
            pl.BlockSpec((1, 1, BLK), lambda i: (i, 0, 0)),
            pl.BlockSpec((2, D), lambda i: (0, 0)),
        ],
        out_shape=[
            jax.ShapeDtypeStruct((S, D), jnp.float32),
            jax.ShapeDtypeStruct((NBLK, 1, BLK), jnp.int32),
            jax.ShapeDtypeStruct((2, D), jnp.float32),
        ],
    )(hs2d, hs2d, hs2d, rw1, rb1, rw2, rb2, dww, dwb)
    return conv, cls3.reshape(S), m10stats


def _register_kernel(reg_ref, m10stats_ref, alpha_ref,
                     regw_ref, regb_ref, reglng_ref, reglnb_ref,
                     wrk_ref, brk_ref, wrv_ref, brv_ref,
                     upd_ref, kr_ref, vr_ref):
    cnt = jnp.maximum(m10stats_ref[1:2, :], 1.0)
    anc_mean = _dot(m10stats_ref[0:1, :], regw_ref[...]) / cnt + regb_ref[...]
    reg = reg_ref[...]
    a = jax.nn.sigmoid(alpha_ref[...])
    upd_ref[...] = _ln(reg + a * (anc_mean - reg), reglng_ref[...], reglnb_ref[...])
    kr_ref[...] = _dot(reg, wrk_ref[...]) + brk_ref[...]
    vr_ref[...] = _dot(reg, wrv_ref[...]) + brv_ref[...]


def _register(reg2d, m10stats, p):
    full = lambda shp: pl.BlockSpec(shp, lambda: (0,) * len(shp))
    return pl.pallas_call(
        _register_kernel,
        in_specs=[full((1, D)), full((2, D)), full((1, 1)),
                  full((D, D)), full((1, D)), full((1, D)), full((1, D)),
                  full((D, D)), full((1, D)), full((D, D)), full((1, D))],
        out_specs=[full((1, D)), full((1, D)), full((1, D))],
        out_shape=[jax.ShapeDtypeStruct((1, D), jnp.float32)] * 3,
    )(reg2d, m10stats, p['alpha'].reshape(1, 1),
      p['reg_w'], p['reg_b'].reshape(1, D),
      p['regln_g'].reshape(1, D), p['regln_b'].reshape(1, D),
      p['wrk'], p['brk'].reshape(1, D), p['wrv'], p['brv'].reshape(1, D))


def _qkv_kernel(n_ref, x_ref, wq_ref, bq_ref, wk_ref, bk_ref, wv_ref, bv_ref,
                q_ref, k_ref, v_ref):
    i = pl.program_id(0)

    @pl.when(i * BLK < n_ref[0])
    def _():
        x = x_ref[...].astype(jnp.bfloat16)
        q_ref[...] = _dot(x, wq_ref[...]) + bq_ref[...]
        k_ref[...] = _dot(x, wk_ref[...]) + bk_ref[...]
        v_ref[...] = _dot(x, wv_ref[...]) + bv_ref[...]


def _qkv(x11, n11, p):
    grid_spec = pltpu.PrefetchScalarGridSpec(
        num_scalar_prefetch=1,
        grid=(NBLK,),
        in_specs=[
            pl.BlockSpec((BLK, D), lambda i, n: (i, 0)),
            pl.BlockSpec((D, D), lambda i, n: (0, 0)),
            pl.BlockSpec((1, D), lambda i, n: (0, 0)),
            pl.BlockSpec((D, D), lambda i, n: (0, 0)),
            pl.BlockSpec((1, D), lambda i, n: (0, 0)),
            pl.BlockSpec((D, D), lambda i, n: (0, 0)),
            pl.BlockSpec((1, D), lambda i, n: (0, 0)),
        ],
        out_specs=[pl.BlockSpec((BLK, D), lambda i, n: (i, 0))] * 3,
    )
    return pl.pallas_call(
        _qkv_kernel,
        grid_spec=grid_spec,
        out_shape=[jax.ShapeDtypeStruct((S, D), jnp.float32)] * 3,
    )(n11, x11, p['wq'].astype(jnp.bfloat16), p['bq'].reshape(1, D),
      p['wk'].astype(jnp.bfloat16), p['bk'].reshape(1, D),
      p['wv'].astype(jnp.bfloat16), p['bv'].reshape(1, D))


def _attn_kernel(n_ref, q_ref, k_ref, v_ref, kr_ref, vr_ref, ao_ref,
                 s_ref, acc_ref):
    qi = pl.program_id(1)
    n = n_ref[0]

    @pl.when(qi * BLK < n)
    def _():
        kidx = jax.lax.broadcasted_iota(jnp.int32, (BLK, S), 1)
        outs = []
        for off in (0, DH):
            q = q_ref[:, off:off + DH]
            qb = q.astype(jnp.bfloat16)
            for j in range(NBLK):
                @pl.when(j * BLK < n)
                def _(j=j, qb=qb, off=off):
                    kj = k_ref[j * BLK:(j + 1) * BLK, off:off + DH]
                    s_ref[:, j * BLK:(j + 1) * BLK] = (
                        _dot(qb, kj.astype(jnp.bfloat16).T) * INV_SCALE)
            s = jnp.where(kidx < n, s_ref[...], -1e9)
            sreg = (q * kr_ref[:, off:off + DH]).sum(
                axis=-1, keepdims=True) * INV_SCALE
            m = jnp.maximum(s.max(axis=-1, keepdims=True), sreg)
            w = jnp.exp(s - m)
            wr = jnp.exp(sreg - m)
            den = w.sum(axis=-1, keepdims=True) + wr
            acc_ref[...] = jnp.zeros((BLK, DH), jnp.float32)
            wb = w.astype(jnp.bfloat16)
            for j in range(NBLK):
                @pl.when(j * BLK < n)
                def _(j=j, wb=wb, off=off):
                    vj = v_ref[j * BLK:(j + 1) * BLK, off:off + DH]
                    acc_ref[...] += _dot(wb[:, j * BLK:(j + 1) * BLK],
                                         vj.astype(jnp.bfloat16))
            outs.append(
                (acc_ref[...] + wr * vr_ref[:, off:off + DH]) / den)
        ao_ref[...] = jnp.concatenate(outs, axis=1)


def _attention(q, k, v, kr, vr, n11):
    grid_spec = pltpu.PrefetchScalarGridSpec(
        num_scalar_prefetch=1,
        grid=(H // 2, NBLK),
        in_specs=[
            pl.BlockSpec((BLK, 2 * DH), lambda h, qi, n: (qi, h)),
            pl.BlockSpec((S, 2 * DH), lambda h, qi, n: (0, h)),
            pl.BlockSpec((S, 2 * DH), lambda h, qi, n: (0, h)),
            pl.BlockSpec((1, 2 * DH), lambda h, qi, n: (0, h)),
            pl.BlockSpec((1, 2 * DH), lambda h, qi, n: (0, h)),
        ],
        out_specs=pl.BlockSpec((BLK, 2 * DH), lambda h, qi, n: (qi, h)),
        scratch_shapes=[pltpu.VMEM((BLK, S), jnp.float32),
                        pltpu.VMEM((BLK, DH), jnp.float32)],
    )
    return pl.pallas_call(
        _attn_kernel,
        grid_spec=grid_spec,
        out_shape=jax.ShapeDtypeStruct((S, D), jnp.float32),
    )(n11, q, k, v, kr, vr)


def _oproj_kernel(n_ref, x_ref, w_ref, b_ref, o_ref):
    @pl.when(pl.program_id(0) * BLK < n_ref[0])
    def _():
        o_ref[...] = _bdot(x_ref[...], w_ref[...]) + b_ref[...]


def _rows_matmul(x, w, b, n):
    grid_spec = pltpu.PrefetchScalarGridSpec(
        num_scalar_prefetch=1,
        grid=(NBLK,),
        in_specs=[
            pl.BlockSpec((BLK, D), lambda i, n: (i, 0)),
            pl.BlockSpec((D, D), lambda i, n: (0, 0)),
            pl.BlockSpec((1, D), lambda i, n: (0, 0)),
        ],
        out_specs=pl.BlockSpec((BLK, D), lambda i, n: (i, 0)),
    )
    return pl.pallas_call(
        _oproj_kernel,
        grid_spec=grid_spec,
        out_shape=jax.ShapeDtypeStruct((S, D), jnp.float32),
    )(n, x, w.astype(jnp.bfloat16), b.reshape(1, D))


def _pw_kernel(n_ref, x_ref, w_ref, b_ref, g_ref, lb_ref, o_ref):
    @pl.when(pl.program_id(0) * BLK < n_ref[0])
    def _():
        y = _bdot(x_ref[...], w_ref[...]) + b_ref[...]
        o_ref[...] = _ln(y, g_ref[...], lb_ref[...])


def _pw(c01, n01, p):
    grid_spec = pltpu.PrefetchScalarGridSpec(
        num_scalar_prefetch=1,
        grid=(NBLK,),
        in_specs=[
            pl.BlockSpec((BLK, D), lambda i, n: (i, 0)),
            pl.BlockSpec((D, D), lambda i, n: (0, 0)),
            pl.BlockSpec((1, D), lambda i, n: (0, 0)),
            pl.BlockSpec((1, D), lambda i, n: (0, 0)),
            pl.BlockSpec((1, D), lambda i, n: (0, 0)),
        ],
        out_specs=pl.BlockSpec((BLK, D), lambda i, n: (i, 0)),
    )
    return pl.pallas_call(
        _pw_kernel,
        grid_spec=grid_spec,
        out_shape=jax.ShapeDtypeStruct((S, D), jnp.float32),
    )(n01, c01, p['pw_w'].astype(jnp.bfloat16), p['pw_b'].reshape(1, D),
      p['cnn_g'].reshape(1, D), p['cnn_b'].reshape(1, D))


def _ffn_kernel(hs_ref, comb_ref, w1_ref, b1_ref, w2_ref, b2_ref,
                g1_ref, lb1_ref, g2_ref, lb2_ref, out_ref):
    x1 = _ln(hs_ref[...] + comb_ref[...], g1_ref[...], lb1_ref[...])
    t = _gelu(_bdot(x1, w1_ref[...]) + b1_ref[...])
    f = _bdot(t, w2_ref[...]) + b2_ref[...]
    out_ref[...] = _ln(x1 + f, g2_ref[...], lb2_ref[...])


def _ffn(hs2d, combined, p):
    return pl.pallas_call(
        _ffn_kernel,
        grid=(NBLK,),
        in_specs=[
            pl.BlockSpec((BLK, D), lambda i: (i, 0)),
            pl.BlockSpec((BLK, D), lambda i: (i, 0)),
            pl.BlockSpec((D, FF), lambda i: (0, 0)),
            pl.BlockSpec((1, FF), lambda i: (0, 0)),
            pl.BlockSpec((FF, D), lambda i: (0, 0)),
            pl.BlockSpec((1, D), lambda i: (0, 0)),
            pl.BlockSpec((1, D), lambda i: (0, 0)),
            pl.BlockSpec((1, D), lambda i: (0, 0)),
            pl.BlockSpec((1, D), lambda i: (0, 0)),
            pl.BlockSpec((1, D), lambda i: (0, 0)),
        ],
        out_specs=pl.BlockSpec((BLK, D), lambda i: (i, 0)),
        out_shape=jax.ShapeDtypeStruct((S, D), jnp.float32),
    )(hs2d, combined, p['ffn_w1'].astype(jnp.bfloat16),
      p['ffn_b1'].reshape(1, FF),
      p['ffn_w2'].astype(jnp.bfloat16), p['ffn_b2'].reshape(1, D),
      p['ln1_g'].reshape(1, D), p['ln1_b'].reshape(1, D),
      p['ln2_g'].reshape(1, D), p['ln2_b'].reshape(1, D))


def _compact(cls):
    mesh = plsc.ScalarSubcoreMesh(axis_name='core', num_cores=1)

    @pl.kernel(
        out_type=[jax.ShapeDtypeStruct((S,), jnp.int32),
                  jax.ShapeDtypeStruct((S,), jnp.int32),
                  jax.ShapeDtypeStruct((S,), jnp.int32),
                  jax.ShapeDtypeStruct((8,), jnp.int32)],
        mesh=mesh,
        scratch_types=[pltpu.SMEM((S,), jnp.int32),
                       pltpu.SMEM((S,), jnp.int32),
                       pltpu.SMEM((S,), jnp.int32),
                       pltpu.SMEM((S,), jnp.int32),
                       pltpu.SMEM((8,), jnp.int32),
                       pltpu.SemaphoreType.DMA],
    )
    def body(cls_hbm, idx01_hbm, idx11_hbm, src_hbm, cnt_hbm,
             cls_s, idx01_s, idx11_s, src_s, cnt_s, sem):
        pltpu.async_copy(cls_hbm, cls_s, sem).wait()
        cnt_s[0] = 0
        cnt_s[1] = 0

        @pl.loop(0, S)
        def _(i):
            c = cls_s[i]
            idx01_s[i] = 0
            idx11_s[i] = 0
            src_s[i] = i

            @pl.when(c == 1)
            def _():
                p = cnt_s[0]
                idx01_s[p] = i
                src_s[i] = S + p
                cnt_s[0] = p + 1

            @pl.when(c == 3)
            def _():
                p = cnt_s[1]
                idx11_s[p] = i
                src_s[i] = 2 * S + p
                cnt_s[1] = p + 1

            @pl.when(c == 4)
            def _():
                src_s[i] = 3 * S

        pltpu.async_copy(idx01_s, idx01_hbm, sem).wait()
        pltpu.async_copy(idx11_s, idx11_hbm, sem).wait()
        pltpu.async_copy(src_s, src_hbm, sem).wait()
        pltpu.async_copy(cnt_s, cnt_hbm, sem).wait()

    idx01, idx11, src, cnts = body(cls)
    return idx01, cnts[0:1], idx11, cnts[1:2], src


_CH = 128
_R = D // _CH
_GW = 128


def _sc_gather(table, idx):
    n_rows = table.shape[0]
    t8 = table.reshape(_R * n_rows, _CH)
    idx8 = (_R * idx[:, None]
            + jnp.arange(_R, dtype=jnp.int32)[None, :]).reshape(1, _R * S)
    mesh = plsc.VectorSubcoreMesh(core_axis_name='core',
                                  subcore_axis_name='subcore')

    @pl.kernel(out_type=jax.ShapeDtypeStruct((_R * S, _CH), jnp.float32),
               mesh=mesh)
    def body(x_hbm, i_hbm, o_hbm):
        def inner(i_vmem, o_vmem):
            pltpu.sync_copy(x_hbm.at[i_vmem.at[0]], o_vmem)

        pltpu.emit_pipeline(
            inner,
            grid=(_R * S // _GW,),
            in_specs=[pl.BlockSpec((1, _GW), lambda i: (0, i))],
            out_specs=[pl.BlockSpec((_GW, _CH), lambda i: (i, 0))],
            core_axis_name=('core', 'subcore'),
            dimension_semantics=(pltpu.PARALLEL,),
        )(i_hbm, o_hbm)

    return body(t8, idx8).reshape(S, D)


def kernel(hidden_states, register_state, params):
    p = params
    hs2d = hidden_states.reshape(S, D)
    reg2d = register_state.reshape(1, D)

    conv, cls, m10stats = _router_conv(hs2d, p)
    upd, kr, vr = _register(reg2d, m10stats, p)

    idx01, n01, idx11, n11, src = _compact(cls)

    x11 = _sc_gather(hs2d, idx11)
    c01 = _sc_gather(conv, idx01)

    q, k, v = _qkv(x11, n11, p)
    ao = _attention(q, k, v, kr, vr, n11)
    attn_cmp = _rows_matmul(ao, p['wo'], p['bo'], n11)
    cnn_cmp = _pw(c01, n01, p)

    pool = jnp.concatenate(
        [hs2d, cnn_cmp, attn_cmp, jnp.zeros((1, D), jnp.float32)], axis=0)
    combined = _sc_gather(pool, src)

    out = _ffn(hs2d, combined, p)
    return out.reshape(1, S, D), upd.reshape(1, D)


# speedup vs baseline: 1.6189x; 1.6189x over previous
import dataclasses
import functools

import jax
import jax.numpy as jnp
import numpy as np
from jax.experimental import pallas as pl
from jax.experimental.pallas import tpu as pltpu
from jax.experimental.pallas import tpu_sc as plsc

D = 1024
H = 16
DH = 64
FF = 4096
KS = 5
S = 2048
BLK = 256
NBLK = S // BLK
INV_SCALE = 1.0 / np.sqrt(DH)

_NSUB = 32
_CHUNK = 16
_NCH = S // _CHUNK
_CPS = _NCH // _NSUB


def _gelu(x):
    return 0.5 * x * (1.0 + jax.lax.erf(x * np.float32(1.0 / np.sqrt(2.0))))


def _ln(x, g, b):
    m = x.mean(-1, keepdims=True)
    v = ((x - m) ** 2).mean(-1, keepdims=True)
    return (x - m) / jnp.sqrt(v + 1e-5) * g + b


def _dot(a, b):
    return jnp.dot(a, b, preferred_element_type=jnp.float32)


def _bdot(a, b):
    return jnp.dot(a.astype(jnp.bfloat16), b,
                   preferred_element_type=jnp.float32)


def _router_cnn_kernel(prev_ref, cur_ref, next_ref, rw1_ref, rb1_ref,
                       rw2_ref, rb2_ref, dww_ref, dwb_ref, pww_ref, pwb_ref,
                       cg_ref, cb_ref,
                       cnn_ref, cls_ref, s01_ref, s11_ref, s00_ref,
                       m10stats_ref):
    i = pl.program_id(0)
    x = cur_ref[...]
    h = _gelu(_dot(x, rw1_ref[...]) + rb1_ref[...])
    lg = _dot(h, rw2_ref[...]) + rb2_ref[...]
    mx = lg.max(axis=-1, keepdims=True)
    e = jnp.exp(lg - mx)
    conf0 = e[:, 0:1] / e.sum(axis=-1, keepdims=True)
    l0, l1, l2 = lg[:, 0:1], lg[:, 1:2], lg[:, 2:3]
    packets = jnp.where(l0 == mx, 0,
                        jnp.where(l1 == mx, 1,
                                  jnp.where(l2 == mx, 2, 3))).astype(jnp.int32)
    m00 = (packets == 0) & (conf0 > 0.99)
    cls = jnp.where(m00, 4, packets)
    cls_ref[0, 0, :] = cls.reshape(1, BLK)[0, :]
    s01_ref[...] = (packets == 1).astype(jnp.float32)
    s11_ref[...] = (packets == 3).astype(jnp.float32)
    s00_ref[...] = m00.astype(jnp.float32)
    m10 = (packets == 2).astype(jnp.float32)
    contrib = (x * m10).sum(axis=0, keepdims=True)
    ccnt = jnp.full((1, D), m10.sum(), jnp.float32)
    stats = jnp.concatenate([contrib, ccnt], axis=0)

    @pl.when(i == 0)
    def _():
        m10stats_ref[...] = stats

    @pl.when(i > 0)
    def _():
        m10stats_ref[...] += stats

    zero2 = jnp.zeros((2, D), jnp.float32)
    top = jnp.where(i > 0, prev_ref[BLK - 2:BLK, :], zero2)
    bot = jnp.where(i < NBLK - 1, next_ref[0:2, :], zero2)
    ext = jnp.concatenate([top, x, bot], axis=0)
    acc = dwb_ref[...]
    for j in range(KS):
        acc = acc + ext[j:j + BLK, :] * dww_ref[j:j + 1, :]
    conv = _gelu(acc)
    y = _bdot(conv, pww_ref[...]) + pwb_ref[...]
    cnn_ref[...] = _ln(y, cg_ref[...], cb_ref[...])


def _router_cnn(hs2d, p):
    dww = jnp.transpose(p['dw_w'][:, 0, :], (1, 0))
    cnn, cls3, s01, s11, s00, m10stats = pl.pallas_call(
        _router_cnn_kernel,
        grid=(NBLK,),
        in_specs=[
            pl.BlockSpec((BLK, D), lambda i: (jnp.maximum(i - 1, 0), 0)),
            pl.BlockSpec((BLK, D), lambda i: (i, 0)),
            pl.BlockSpec((BLK, D), lambda i: (jnp.minimum(i + 1, NBLK - 1), 0)),
            pl.BlockSpec((D, 64), lambda i: (0, 0)),
            pl.BlockSpec((1, 64), lambda i: (0, 0)),
            pl.BlockSpec((64, 4), lambda i: (0, 0)),
            pl.BlockSpec((1, 4), lambda i: (0, 0)),
            pl.BlockSpec((KS, D), lambda i: (0, 0)),
            pl.BlockSpec((1, D), lambda i: (0, 0)),
            pl.BlockSpec((D, D), lambda i: (0, 0)),
            pl.BlockSpec((1, D), lambda i: (0, 0)),
            pl.BlockSpec((1, D), lambda i: (0, 0)),
            pl.BlockSpec((1, D), lambda i: (0, 0)),
        ],
        out_specs=[
            pl.BlockSpec((BLK, D), lambda i: (i, 0)),
            pl.BlockSpec((1, 1, BLK), lambda i: (i, 0, 0)),
            pl.BlockSpec((BLK, 1), lambda i: (i, 0)),
            pl.BlockSpec((BLK, 1), lambda i: (i, 0)),
            pl.BlockSpec((BLK, 1), lambda i: (i, 0)),
            pl.BlockSpec((2, D), lambda i: (0, 0)),
        ],
        out_shape=[
            jax.ShapeDtypeStruct((S, D), jnp.float32),
            jax.ShapeDtypeStruct((NBLK, 1, BLK), jnp.int32),
            jax.ShapeDtypeStruct((S, 1), jnp.float32),
            jax.ShapeDtypeStruct((S, 1), jnp.float32),
            jax.ShapeDtypeStruct((S, 1), jnp.float32),
            jax.ShapeDtypeStruct((2, D), jnp.float32),
        ],
    )(hs2d, hs2d, hs2d, p['r_w1'], p['r_b1'].reshape(1, 64),
      p['r_w2'], p['r_b2'].reshape(1, 4), dww, p['dw_b'].reshape(1, D),
      p['pw_w'].astype(jnp.bfloat16), p['pw_b'].reshape(1, D),
      p['cnn_g'].reshape(1, D), p['cnn_b'].reshape(1, D))
    return cnn, cls3.reshape(S), s01, s11, s00, m10stats


def _register_kernel(reg_ref, m10stats_ref, alpha_ref,
                     regw_ref, regb_ref, reglng_ref, reglnb_ref,
                     wrk_ref, brk_ref, wrv_ref, brv_ref,
                     upd_ref, kr_ref, vr_ref):
    cnt = jnp.maximum(m10stats_ref[1:2, :], 1.0)
    anc_mean = _dot(m10stats_ref[0:1, :], regw_ref[...]) / cnt + regb_ref[...]
    reg = reg_ref[...]
    a = jax.nn.sigmoid(alpha_ref[...])
    upd_ref[...] = _ln(reg + a * (anc_mean - reg), reglng_ref[...], reglnb_ref[...])
    kr_ref[...] = _dot(reg, wrk_ref[...]) + brk_ref[...]
    vr_ref[...] = _dot(reg, wrv_ref[...]) + brv_ref[...]


def _register(reg2d, m10stats, p):
    full = lambda shp: pl.BlockSpec(shp, lambda: (0,) * len(shp))
    return pl.pallas_call(
        _register_kernel,
        in_specs=[full((1, D)), full((2, D)), full((1, 1)),
                  full((D, D)), full((1, D)), full((1, D)), full((1, D)),
                  full((D, D)), full((1, D)), full((D, D)), full((1, D))],
        out_specs=[full((1, D)), full((1, D)), full((1, D))],
        out_shape=[jax.ShapeDtypeStruct((1, D), jnp.float32)] * 3,
    )(reg2d, m10stats, p['alpha'].reshape(1, 1),
      p['reg_w'], p['reg_b'].reshape(1, D),
      p['regln_g'].reshape(1, D), p['regln_b'].reshape(1, D),
      p['wrk'], p['brk'].reshape(1, D), p['wrv'], p['brv'].reshape(1, D))


def _compact(cls):
    mesh = plsc.ScalarSubcoreMesh(axis_name='core', num_cores=1)

    @pl.kernel(
        out_type=[jax.ShapeDtypeStruct((S,), jnp.int32),
                  jax.ShapeDtypeStruct((S,), jnp.int32),
                  jax.ShapeDtypeStruct((16,), jnp.int32)],
        mesh=mesh,
        scratch_types=[pltpu.SMEM((S,), jnp.int32),
                       pltpu.SMEM((S,), jnp.int32),
                       pltpu.SMEM((S,), jnp.int32),
                       pltpu.SMEM((16,), jnp.int32),
                       pltpu.SemaphoreType.DMA],
    )
    def body(cls_hbm, idxg_hbm, idxs_hbm, cnt_hbm,
             cls_s, idxg_s, idxs_s, cnt_s, sem):
        pltpu.async_copy(cls_hbm, cls_s, sem).wait()

        @pl.loop(0, 16)
        def _(i):
            cnt_s[i] = 0

        @pl.loop(0, S)
        def _(i):
            idxg_s[i] = 0
            idxs_s[i] = S

            @pl.when(cls_s[i] == 3)
            def _():
                p = cnt_s[0]
                idxg_s[p] = i
                idxs_s[p] = i
                cnt_s[0] = p + 1

        pltpu.async_copy(idxg_s, idxg_hbm, sem).wait()
        pltpu.async_copy(idxs_s, idxs_hbm, sem).wait()
        pltpu.async_copy(cnt_s, cnt_hbm, sem).wait()

    idxg, idxs, cnts = body(cls)
    return idxg, idxs, cnts


def _sc_vec_params():
    cp = pltpu.CompilerParams()
    if "needs_layout_passes" in pltpu.CompilerParams.__dataclass_fields__:
        cp = dataclasses.replace(cp, needs_layout_passes=False)
    return cp


def _sc_gather(table, idx, cnts):
    mesh = plsc.VectorSubcoreMesh(core_axis_name='c', subcore_axis_name='s')

    @pl.kernel(
        out_type=jax.ShapeDtypeStruct((S, D), jnp.float32),
        mesh=mesh,
        compiler_params=_sc_vec_params(),
        scratch_types=[pltpu.VMEM((_CHUNK,), jnp.int32),
                       pltpu.VMEM((_CHUNK, D), jnp.float32),
                       pltpu.VMEM((16,), jnp.int32),
                       pltpu.SemaphoreType.DMA],
    )
    def body(table_hbm, idx_hbm, cnt_hbm, out_hbm, idx_v, rows_v, cnt_v, sem):
        pltpu.sync_copy(cnt_hbm, cnt_v)
        n = jnp.max(cnt_v[...])
        wid = jax.lax.axis_index('s') * 2 + jax.lax.axis_index('c')
        for c in range(_CPS):
            j = wid + c * _NSUB

            @pl.when(j * _CHUNK < n)
            def _(j=j):
                base = j * _CHUNK
                pltpu.sync_copy(idx_hbm.at[pl.ds(base, _CHUNK)], idx_v)
                pltpu.async_copy(table_hbm.at[idx_v], rows_v, sem).wait()
                pltpu.sync_copy(rows_v, out_hbm.at[pl.ds(base, _CHUNK)])

    return body(table, idx, cnts)


def _sc_scatter(rows, idx, cnts):
    mesh = plsc.VectorSubcoreMesh(core_axis_name='c', subcore_axis_name='s')

    @pl.kernel(
        out_type=jax.ShapeDtypeStruct((S + _CHUNK, D), jnp.float32),
        mesh=mesh,
        compiler_params=_sc_vec_params(),
        scratch_types=[pltpu.VMEM((_CHUNK,), jnp.int32),
                       pltpu.VMEM((_CHUNK, D), jnp.float32),
                       pltpu.VMEM((16,), jnp.int32),
                       pltpu.SemaphoreType.DMA],
    )
    def body(rows_hbm, idx_hbm, cnt_hbm, out_hbm, idx_v, rows_v, cnt_v, sem):
        pltpu.sync_copy(cnt_hbm, cnt_v)
        n = jnp.max(cnt_v[...])
        wid = jax.lax.axis_index('s') * 2 + jax.lax.axis_index('c')
        for c in range(_CPS):
            j = wid + c * _NSUB

            @pl.when(j * _CHUNK < n)
            def _(j=j):
                base = j * _CHUNK
                pltpu.sync_copy(idx_hbm.at[pl.ds(base, _CHUNK)], idx_v)
                pltpu.sync_copy(rows_hbm.at[pl.ds(base, _CHUNK)], rows_v)
                pltpu.async_copy(rows_v, out_hbm.at[idx_v], sem).wait()

    return body(rows, idx, cnts)


def _qkv_kernel(n_ref, x_ref, wq_ref, bq_ref, wk_ref, bk_ref, wv_ref, bv_ref,
                q_ref, k_ref, v_ref):
    i = pl.program_id(0)

    @pl.when(i * BLK < n_ref[0])
    def _():
        ridx = i * BLK + jax.lax.broadcasted_iota(jnp.int32, (BLK, 1), 0)
        x = jnp.where(ridx < n_ref[0], x_ref[...], 0.0).astype(jnp.bfloat16)
        q_ref[...] = _dot(x, wq_ref[...]) + bq_ref[...]
        k_ref[...] = _dot(x, wk_ref[...]) + bk_ref[...]
        v_ref[...] = _dot(x, wv_ref[...]) + bv_ref[...]


def _qkv(x11, n11, p):
    grid_spec = pltpu.PrefetchScalarGridSpec(
        num_scalar_prefetch=1,
        grid=(NBLK,),
        in_specs=[
            pl.BlockSpec((BLK, D), lambda i, n: (i, 0)),
            pl.BlockSpec((D, D), lambda i, n: (0, 0)),
            pl.BlockSpec((1, D), lambda i, n: (0, 0)),
            pl.BlockSpec((D, D), lambda i, n: (0, 0)),
            pl.BlockSpec((1, D), lambda i, n: (0, 0)),
            pl.BlockSpec((D, D), lambda i, n: (0, 0)),
            pl.BlockSpec((1, D), lambda i, n: (0, 0)),
        ],
        out_specs=[pl.BlockSpec((BLK, D), lambda i, n: (i, 0))] * 3,
    )
    return pl.pallas_call(
        _qkv_kernel,
        grid_spec=grid_spec,
        out_shape=[jax.ShapeDtypeStruct((S, D), jnp.float32)] * 3,
    )(n11, x11, p['wq'].astype(jnp.bfloat16), p['bq'].reshape(1, D),
      p['wk'].astype(jnp.bfloat16), p['bk'].reshape(1, D),
      p['wv'].astype(jnp.bfloat16), p['bv'].reshape(1, D))


def _attn_kernel(n_ref, q_ref, k_ref, v_ref, kr_ref, vr_ref, ao_ref,
                 s_ref, acc_ref):
    qi = pl.program_id(1)
    n = n_ref[0]

    @pl.when(qi * BLK < n)
    def _():
        kidx = jax.lax.broadcasted_iota(jnp.int32, (BLK, S), 1)
        outs = []
        for off in (0, DH):
            q = q_ref[:, off:off + DH]
            qb = q.astype(jnp.bfloat16)
            for j in range(NBLK):
                @pl.when(j * BLK < n)
                def _(j=j, qb=qb, off=off):
                    kj = k_ref[j * BLK:(j + 1) * BLK, off:off + DH]
                    s_ref[:, j * BLK:(j + 1) * BLK] = (
                        _dot(qb, kj.astype(jnp.bfloat16).T) * INV_SCALE)
            s = jnp.where(kidx < n, s_ref[...], -1e9)
            sreg = (q * kr_ref[:, off:off + DH]).sum(
                axis=-1, keepdims=True) * INV_SCALE
            m = jnp.maximum(s.max(axis=-1, keepdims=True), sreg)
            w = jnp.exp(s - m)
            wr = jnp.exp(sreg - m)
            den = w.sum(axis=-1, keepdims=True) + wr
            acc_ref[...] = jnp.zeros((BLK, DH), jnp.float32)
            wb = w.astype(jnp.bfloat16)
            for j in range(NBLK):
                @pl.when(j * BLK < n)
                def _(j=j, wb=wb, off=off):
                    vj = v_ref[j * BLK:(j + 1) * BLK, off:off + DH]
                    acc_ref[...] += _dot(wb[:, j * BLK:(j + 1) * BLK],
                                         vj.astype(jnp.bfloat16))
            outs.append(
                (acc_ref[...] + wr * vr_ref[:, off:off + DH]) / den)
        ao_ref[...] = jnp.concatenate(outs, axis=1)


def _attention(q, k, v, kr, vr, n11):
    grid_spec = pltpu.PrefetchScalarGridSpec(
        num_scalar_prefetch=1,
        grid=(H // 2, NBLK),
        in_specs=[
            pl.BlockSpec((BLK, 2 * DH), lambda h, qi, n: (qi, h)),
            pl.BlockSpec((S, 2 * DH), lambda h, qi, n: (0, h)),
            pl.BlockSpec((S, 2 * DH), lambda h, qi, n: (0, h)),
            pl.BlockSpec((1, 2 * DH), lambda h, qi, n: (0, h)),
            pl.BlockSpec((1, 2 * DH), lambda h, qi, n: (0, h)),
        ],
        out_specs=pl.BlockSpec((BLK, 2 * DH), lambda h, qi, n: (qi, h)),
        scratch_shapes=[pltpu.VMEM((BLK, S), jnp.float32),
                        pltpu.VMEM((BLK, DH), jnp.float32)],
    )
    return pl.pallas_call(
        _attn_kernel,
        grid_spec=grid_spec,
        out_shape=jax.ShapeDtypeStruct((S, D), jnp.float32),
    )(n11, q, k, v, kr, vr)


def _oproj_kernel(n_ref, x_ref, w_ref, b_ref, o_ref):
    @pl.when(pl.program_id(0) * BLK < n_ref[0])
    def _():
        o_ref[...] = _bdot(x_ref[...], w_ref[...]) + b_ref[...]


def _oproj(x, n, p):
    grid_spec = pltpu.PrefetchScalarGridSpec(
        num_scalar_prefetch=1,
        grid=(NBLK,),
        in_specs=[
            pl.BlockSpec((BLK, D), lambda i, n: (i, 0)),
            pl.BlockSpec((D, D), lambda i, n: (0, 0)),
            pl.BlockSpec((1, D), lambda i, n: (0, 0)),
        ],
        out_specs=pl.BlockSpec((BLK, D), lambda i, n: (i, 0)),
    )
    return pl.pallas_call(
        _oproj_kernel,
        grid_spec=grid_spec,
        out_shape=jax.ShapeDtypeStruct((S, D), jnp.float32),
    )(n, x, p['wo'].astype(jnp.bfloat16), p['bo'].reshape(1, D))


def _ffn_kernel(hs_ref, cnn_ref, attn_ref, s01_ref, s11_ref, s00_ref,
                w1_ref, b1_ref, w2_ref, b2_ref,
                g1_ref, lb1_ref, g2_ref, lb2_ref, out_ref):
    hs = hs_ref[...]
    combined = jnp.where(s00_ref[...] > 0.5, 0.0, hs)
    combined = jnp.where(s01_ref[...] > 0.5, cnn_ref[...], combined)
    combined = jnp.where(s11_ref[...] > 0.5, attn_ref[...], combined)
    x1 = _ln(hs + combined, g1_ref[...], lb1_ref[...])
    t = _gelu(_bdot(x1, w1_ref[...]) + b1_ref[...])
    f = _bdot(t, w2_ref[...]) + b2_ref[...]
    out_ref[...] = _ln(x1 + f, g2_ref[...], lb2_ref[...])


def _ffn(hs2d, cnn, attn, s01, s11, s00, p):
    return pl.pallas_call(
        _ffn_kernel,
        grid=(NBLK,),
        in_specs=[
            pl.BlockSpec((BLK, D), lambda i: (i, 0)),
            pl.BlockSpec((BLK, D), lambda i: (i, 0)),
            pl.BlockSpec((BLK, D), lambda i: (i, 0)),
            pl.BlockSpec((BLK, 1), lambda i: (i, 0)),
            pl.BlockSpec((BLK, 1), lambda i: (i, 0)),
            pl.BlockSpec((BLK, 1), lambda i: (i, 0)),
            pl.BlockSpec((D, FF), lambda i: (0, 0)),
            pl.BlockSpec((1, FF), lambda i: (0, 0)),
            pl.BlockSpec((FF, D), lambda i: (0, 0)),
            pl.BlockSpec((1, D), lambda i: (0, 0)),
            pl.BlockSpec((1, D), lambda i: (0, 0)),
            pl.BlockSpec((1, D), lambda i: (0, 0)),
            pl.BlockSpec((1, D), lambda i: (0, 0)),
            pl.BlockSpec((1, D), lambda i: (0, 0)),
        ],
        out_specs=pl.BlockSpec((BLK, D), lambda i: (i, 0)),
        out_shape=jax.ShapeDtypeStruct((S, D), jnp.float32),
    )(hs2d, cnn, attn, s01, s11, s00,
      p['ffn_w1'].astype(jnp.bfloat16), p['ffn_b1'].reshape(1, FF),
      p['ffn_w2'].astype(jnp.bfloat16), p['ffn_b2'].reshape(1, D),
      p['ln1_g'].reshape(1, D), p['ln1_b'].reshape(1, D),
      p['ln2_g'].reshape(1, D), p['ln2_b'].reshape(1, D))


def kernel(hidden_states, register_state, params):
    p = params
    hs2d = hidden_states.reshape(S, D)
    reg2d = register_state.reshape(1, D)

    cnn, cls, s01, s11, s00, m10stats = _router_cnn(hs2d, p)
    upd, kr, vr = _register(reg2d, m10stats, p)

    idxg, idxs, cnts = _compact(cls)
    n11 = cnts[0:1]

    x11 = _sc_gather(hs2d, idxg, cnts)
    q, k, v = _qkv(x11, n11, p)
    ao = _attention(q, k, v, kr, vr, n11)
    attn_cmp = _oproj(ao, n11, p)
    attn_pos = _sc_scatter(attn_cmp, idxs, cnts)

    out = _ffn(hs2d, cnn, attn_pos, s01, s11, s00, p)
    return out.reshape(1, S, D), upd.reshape(1, D)


# speedup vs baseline: 1.6225x; 1.0022x over previous
import dataclasses
import functools

import jax
import jax.numpy as jnp
import numpy as np
from jax.experimental import pallas as pl
from jax.experimental.pallas import tpu as pltpu
from jax.experimental.pallas import tpu_sc as plsc

D = 1024
H = 16
DH = 64
FF = 4096
KS = 5
S = 2048
BLK = 256
NBLK = S // BLK
INV_SCALE = 1.0 / np.sqrt(DH)

_NSUB = 32
_CHUNK = 16
_NCH = S // _CHUNK
_CPS = _NCH // _NSUB


def _gelu(x):
    return 0.5 * x * (1.0 + jax.lax.erf(x * np.float32(1.0 / np.sqrt(2.0))))


def _ln(x, g, b):
    m = x.mean(-1, keepdims=True)
    v = ((x - m) ** 2).mean(-1, keepdims=True)
    return (x - m) / jnp.sqrt(v + 1e-5) * g + b


def _dot(a, b):
    return jnp.dot(a, b, preferred_element_type=jnp.float32)


def _bdot(a, b):
    return jnp.dot(a.astype(jnp.bfloat16), b,
                   preferred_element_type=jnp.float32)


def _router_cnn_kernel(prev_ref, cur_ref, next_ref, rw1_ref, rb1_ref,
                       rw2_ref, rb2_ref, dww_ref, dwb_ref, pww_ref, pwb_ref,
                       cg_ref, cb_ref,
                       cnn_ref, cls_ref, s01_ref, s11_ref, s00_ref,
                       m10stats_ref):
    i = pl.program_id(0)
    x = cur_ref[...]
    h = _gelu(_dot(x, rw1_ref[...]) + rb1_ref[...])
    lg = _dot(h, rw2_ref[...]) + rb2_ref[...]
    mx = lg.max(axis=-1, keepdims=True)
    e = jnp.exp(lg - mx)
    conf0 = e[:, 0:1] / e.sum(axis=-1, keepdims=True)
    l0, l1, l2 = lg[:, 0:1], lg[:, 1:2], lg[:, 2:3]
    packets = jnp.where(l0 == mx, 0,
                        jnp.where(l1 == mx, 1,
                                  jnp.where(l2 == mx, 2, 3))).astype(jnp.int32)
    m00 = (packets == 0) & (conf0 > 0.99)
    cls = jnp.where(m00, 4, packets)
    cls_ref[0, 0, :] = cls.reshape(1, BLK)[0, :]
    s01_ref[...] = (packets == 1).astype(jnp.float32)
    s11_ref[...] = (packets == 3).astype(jnp.float32)
    s00_ref[...] = m00.astype(jnp.float32)
    m10 = (packets == 2).astype(jnp.float32)
    contrib = (x * m10).sum(axis=0, keepdims=True)
    ccnt = jnp.full((1, D), m10.sum(), jnp.float32)
    stats = jnp.concatenate([contrib, ccnt], axis=0)

    @pl.when(i == 0)
    def _():
        m10stats_ref[...] = stats

    @pl.when(i > 0)
    def _():
        m10stats_ref[...] += stats

    zero2 = jnp.zeros((2, D), jnp.float32)
    top = jnp.where(i > 0, prev_ref[BLK - 2:BLK, :], zero2)
    bot = jnp.where(i < NBLK - 1, next_ref[0:2, :], zero2)
    ext = jnp.concatenate([top, x, bot], axis=0)
    acc = dwb_ref[...]
    for j in range(KS):
        acc = acc + ext[j:j + BLK, :] * dww_ref[j:j + 1, :]
    conv = _gelu(acc)
    y = _bdot(conv, pww_ref[...]) + pwb_ref[...]
    cnn_ref[...] = _ln(y, cg_ref[...], cb_ref[...])


def _router_cnn(hs2d, p):
    dww = jnp.transpose(p['dw_w'][:, 0, :], (1, 0))
    cnn, cls3, s01, s11, s00, m10stats = pl.pallas_call(
        _router_cnn_kernel,
        grid=(NBLK,),
        in_specs=[
            pl.BlockSpec((BLK, D), lambda i: (jnp.maximum(i - 1, 0), 0)),
            pl.BlockSpec((BLK, D), lambda i: (i, 0)),
            pl.BlockSpec((BLK, D), lambda i: (jnp.minimum(i + 1, NBLK - 1), 0)),
            pl.BlockSpec((D, 64), lambda i: (0, 0)),
            pl.BlockSpec((1, 64), lambda i: (0, 0)),
            pl.BlockSpec((64, 4), lambda i: (0, 0)),
            pl.BlockSpec((1, 4), lambda i: (0, 0)),
            pl.BlockSpec((KS, D), lambda i: (0, 0)),
            pl.BlockSpec((1, D), lambda i: (0, 0)),
            pl.BlockSpec((D, D), lambda i: (0, 0)),
            pl.BlockSpec((1, D), lambda i: (0, 0)),
            pl.BlockSpec((1, D), lambda i: (0, 0)),
            pl.BlockSpec((1, D), lambda i: (0, 0)),
        ],
        out_specs=[
            pl.BlockSpec((BLK, D), lambda i: (i, 0)),
            pl.BlockSpec((1, 1, BLK), lambda i: (i, 0, 0)),
            pl.BlockSpec((BLK, 1), lambda i: (i, 0)),
            pl.BlockSpec((BLK, 1), lambda i: (i, 0)),
            pl.BlockSpec((BLK, 1), lambda i: (i, 0)),
            pl.BlockSpec((2, D), lambda i: (0, 0)),
        ],
        out_shape=[
            jax.ShapeDtypeStruct((S, D), jnp.float32),
            jax.ShapeDtypeStruct((NBLK, 1, BLK), jnp.int32),
            jax.ShapeDtypeStruct((S, 1), jnp.float32),
            jax.ShapeDtypeStruct((S, 1), jnp.float32),
            jax.ShapeDtypeStruct((S, 1), jnp.float32),
            jax.ShapeDtypeStruct((2, D), jnp.float32),
        ],
    )(hs2d, hs2d, hs2d, p['r_w1'], p['r_b1'].reshape(1, 64),
      p['r_w2'], p['r_b2'].reshape(1, 4), dww, p['dw_b'].reshape(1, D),
      p['pw_w'].astype(jnp.bfloat16), p['pw_b'].reshape(1, D),
      p['cnn_g'].reshape(1, D), p['cnn_b'].reshape(1, D))
    return cnn, cls3.reshape(S), s01, s11, s00, m10stats


def _register_kernel(reg_ref, m10stats_ref, alpha_ref,
                     regw_ref, regb_ref, reglng_ref, reglnb_ref,
                     wrk_ref, brk_ref, wrv_ref, brv_ref,
                     upd_ref, kr_ref, vr_ref):
    cnt = jnp.maximum(m10stats_ref[1:2, :], 1.0)
    anc_mean = _dot(m10stats_ref[0:1, :], regw_ref[...]) / cnt + regb_ref[...]
    reg = reg_ref[...]
    a = jax.nn.sigmoid(alpha_ref[...])
    upd_ref[...] = _ln(reg + a * (anc_mean - reg), reglng_ref[...], reglnb_ref[...])
    kr_ref[...] = _dot(reg, wrk_ref[...]) + brk_ref[...]
    vr_ref[...] = _dot(reg, wrv_ref[...]) + brv_ref[...]


def _register(reg2d, m10stats, p):
    full = lambda shp: pl.BlockSpec(shp, lambda: (0,) * len(shp))
    return pl.pallas_call(
        _register_kernel,
        in_specs=[full((1, D)), full((2, D)), full((1, 1)),
                  full((D, D)), full((1, D)), full((1, D)), full((1, D)),
                  full((D, D)), full((1, D)), full((D, D)), full((1, D))],
        out_specs=[full((1, D)), full((1, D)), full((1, D))],
        out_shape=[jax.ShapeDtypeStruct((1, D), jnp.float32)] * 3,
    )(reg2d, m10stats, p['alpha'].reshape(1, 1),
      p['reg_w'], p['reg_b'].reshape(1, D),
      p['regln_g'].reshape(1, D), p['regln_b'].reshape(1, D),
      p['wrk'], p['brk'].reshape(1, D), p['wrv'], p['brv'].reshape(1, D))


def _compact(cls):
    mesh = plsc.ScalarSubcoreMesh(axis_name='core', num_cores=1)

    @pl.kernel(
        out_type=[jax.ShapeDtypeStruct((S,), jnp.int32),
                  jax.ShapeDtypeStruct((S,), jnp.int32),
                  jax.ShapeDtypeStruct((16,), jnp.int32)],
        mesh=mesh,
        scratch_types=[pltpu.SMEM((S,), jnp.int32),
                       pltpu.SMEM((S,), jnp.int32),
                       pltpu.SMEM((S,), jnp.int32),
                       pltpu.SMEM((16,), jnp.int32),
                       pltpu.SemaphoreType.DMA],
    )
    def body(cls_hbm, idxg_hbm, idxs_hbm, cnt_hbm,
             cls_s, idxg_s, idxs_s, cnt_s, sem):
        pltpu.async_copy(cls_hbm, cls_s, sem).wait()

        @pl.loop(0, 16)
        def _(i):
            cnt_s[i] = 0

        @pl.loop(0, S)
        def _(i):
            idxg_s[i] = 0
            idxs_s[i] = S

            @pl.when(cls_s[i] == 3)
            def _():
                p = cnt_s[0]
                idxg_s[p] = i
                idxs_s[p] = i
                cnt_s[0] = p + 1

        pltpu.async_copy(idxg_s, idxg_hbm, sem).wait()
        pltpu.async_copy(idxs_s, idxs_hbm, sem).wait()
        pltpu.async_copy(cnt_s, cnt_hbm, sem).wait()

    idxg, idxs, cnts = body(cls)
    return idxg, idxs, cnts


def _sc_vec_params():
    cp = pltpu.CompilerParams()
    if "needs_layout_passes" in pltpu.CompilerParams.__dataclass_fields__:
        cp = dataclasses.replace(cp, needs_layout_passes=False)
    return cp


def _sc_gather(table, idx, cnts):
    mesh = plsc.VectorSubcoreMesh(core_axis_name='c', subcore_axis_name='s')

    @pl.kernel(
        out_type=jax.ShapeDtypeStruct((S, D), jnp.float32),
        mesh=mesh,
        compiler_params=_sc_vec_params(),
        scratch_types=[pltpu.VMEM((_CHUNK,), jnp.int32),
                       pltpu.VMEM((_CHUNK, D), jnp.float32),
                       pltpu.VMEM((16,), jnp.int32),
                       pltpu.SemaphoreType.DMA],
    )
    def body(table_hbm, idx_hbm, cnt_hbm, out_hbm, idx_v, rows_v, cnt_v, sem):
        pltpu.sync_copy(cnt_hbm, cnt_v)
        n = jnp.max(cnt_v[...])
        wid = jax.lax.axis_index('s') * 2 + jax.lax.axis_index('c')
        for c in range(_CPS):
            j = wid + c * _NSUB

            @pl.when(j * _CHUNK < n)
            def _(j=j):
                base = j * _CHUNK
                pltpu.sync_copy(idx_hbm.at[pl.ds(base, _CHUNK)], idx_v)
                pltpu.async_copy(table_hbm.at[idx_v], rows_v, sem).wait()
                pltpu.sync_copy(rows_v, out_hbm.at[pl.ds(base, _CHUNK)])

    return body(table, idx, cnts)


def _sc_scatter(rows, idx, cnts):
    mesh = plsc.VectorSubcoreMesh(core_axis_name='c', subcore_axis_name='s')

    @pl.kernel(
        out_type=jax.ShapeDtypeStruct((S + _CHUNK, D), jnp.float32),
        mesh=mesh,
        compiler_params=_sc_vec_params(),
        scratch_types=[pltpu.VMEM((_CHUNK,), jnp.int32),
                       pltpu.VMEM((_CHUNK, D), jnp.float32),
                       pltpu.VMEM((16,), jnp.int32),
                       pltpu.SemaphoreType.DMA],
    )
    def body(rows_hbm, idx_hbm, cnt_hbm, out_hbm, idx_v, rows_v, cnt_v, sem):
        pltpu.sync_copy(cnt_hbm, cnt_v)
        n = jnp.max(cnt_v[...])
        wid = jax.lax.axis_index('s') * 2 + jax.lax.axis_index('c')
        for c in range(_CPS):
            j = wid + c * _NSUB

            @pl.when(j * _CHUNK < n)
            def _(j=j):
                base = j * _CHUNK
                pltpu.sync_copy(idx_hbm.at[pl.ds(base, _CHUNK)], idx_v)
                pltpu.sync_copy(rows_hbm.at[pl.ds(base, _CHUNK)], rows_v)
                pltpu.async_copy(rows_v, out_hbm.at[idx_v], sem).wait()

    return body(rows, idx, cnts)


def _qkv_kernel(n_ref, x_ref, wq_ref, bq_ref, wk_ref, bk_ref, wv_ref, bv_ref,
                q_ref, k_ref, v_ref):
    i = pl.program_id(0)

    @pl.when(i * BLK < n_ref[0])
    def _():
        ridx = i * BLK + jax.lax.broadcasted_iota(jnp.int32, (BLK, 1), 0)
        x = jnp.where(ridx < n_ref[0], x_ref[...], 0.0).astype(jnp.bfloat16)
        q_ref[...] = _dot(x, wq_ref[...]) + bq_ref[...]
        k_ref[...] = _dot(x, wk_ref[...]) + bk_ref[...]
        v_ref[...] = _dot(x, wv_ref[...]) + bv_ref[...]


def _qkv(x11, n11, p):
    grid_spec = pltpu.PrefetchScalarGridSpec(
        num_scalar_prefetch=1,
        grid=(NBLK,),
        in_specs=[
            pl.BlockSpec((BLK, D), lambda i, n: (i, 0)),
            pl.BlockSpec((D, D), lambda i, n: (0, 0)),
            pl.BlockSpec((1, D), lambda i, n: (0, 0)),
            pl.BlockSpec((D, D), lambda i, n: (0, 0)),
            pl.BlockSpec((1, D), lambda i, n: (0, 0)),
            pl.BlockSpec((D, D), lambda i, n: (0, 0)),
            pl.BlockSpec((1, D), lambda i, n: (0, 0)),
        ],
        out_specs=[pl.BlockSpec((BLK, D), lambda i, n: (i, 0))] * 3,
    )
    return pl.pallas_call(
        _qkv_kernel,
        grid_spec=grid_spec,
        out_shape=[jax.ShapeDtypeStruct((S, D), jnp.float32)] * 3,
    )(n11, x11, p['wq'].astype(jnp.bfloat16), p['bq'].reshape(1, D),
      p['wk'].astype(jnp.bfloat16), p['bk'].reshape(1, D),
      p['wv'].astype(jnp.bfloat16), p['bv'].reshape(1, D))


def _attn_kernel(n_ref, q_ref, k_ref, v_ref, kr_ref, vr_ref, ao_ref,
                 s_ref, m_ref, den_ref, acc_ref):
    qi = pl.program_id(1)
    n = n_ref[0]

    @pl.when(qi * BLK < n)
    def _():
        outs = []
        for off in (0, DH):
            q = q_ref[:, off:off + DH]
            qb = q.astype(jnp.bfloat16)
            sreg = (q * kr_ref[:, off:off + DH]).sum(
                axis=-1, keepdims=True) * INV_SCALE
            m_ref[...] = sreg
            for j in range(NBLK):
                @pl.when(j * BLK < n)
                def _(j=j, qb=qb, off=off):
                    kj = k_ref[j * BLK:(j + 1) * BLK, off:off + DH]
                    sj = _dot(qb, kj.astype(jnp.bfloat16).T) * INV_SCALE
                    kidx = j * BLK + jax.lax.broadcasted_iota(
                        jnp.int32, (BLK, BLK), 1)
                    sj = jnp.where(kidx < n, sj, -1e9)
                    s_ref[:, j * BLK:(j + 1) * BLK] = sj
                    m_ref[...] = jnp.maximum(
                        m_ref[...], sj.max(axis=-1, keepdims=True))
            m = m_ref[...]
            wr = jnp.exp(sreg - m)
            den_ref[...] = wr
            acc_ref[...] = jnp.zeros((BLK, DH), jnp.float32)
            for j in range(NBLK):
                @pl.when(j * BLK < n)
                def _(j=j, m=m, off=off):
                    wj = jnp.exp(s_ref[:, j * BLK:(j + 1) * BLK] - m)
                    den_ref[...] += wj.sum(axis=-1, keepdims=True)
                    vj = v_ref[j * BLK:(j + 1) * BLK, off:off + DH]
                    acc_ref[...] += _dot(wj.astype(jnp.bfloat16),
                                         vj.astype(jnp.bfloat16))
            outs.append(
                (acc_ref[...] + wr * vr_ref[:, off:off + DH]) / den_ref[...])
        ao_ref[...] = jnp.concatenate(outs, axis=1)


def _attention(q, k, v, kr, vr, n11):
    grid_spec = pltpu.PrefetchScalarGridSpec(
        num_scalar_prefetch=1,
        grid=(H // 2, NBLK),
        in_specs=[
            pl.BlockSpec((BLK, 2 * DH), lambda h, qi, n: (qi, h)),
            pl.BlockSpec((S, 2 * DH), lambda h, qi, n: (0, h)),
            pl.BlockSpec((S, 2 * DH), lambda h, qi, n: (0, h)),
            pl.BlockSpec((1, 2 * DH), lambda h, qi, n: (0, h)),
            pl.BlockSpec((1, 2 * DH), lambda h, qi, n: (0, h)),
        ],
        out_specs=pl.BlockSpec((BLK, 2 * DH), lambda h, qi, n: (qi, h)),
        scratch_shapes=[pltpu.VMEM((BLK, S), jnp.float32),
                        pltpu.VMEM((BLK, 1), jnp.float32),
                        pltpu.VMEM((BLK, 1), jnp.float32),
                        pltpu.VMEM((BLK, DH), jnp.float32)],
    )
    return pl.pallas_call(
        _attn_kernel,
        grid_spec=grid_spec,
        out_shape=jax.ShapeDtypeStruct((S, D), jnp.float32),
    )(n11, q, k, v, kr, vr)


def _oproj_kernel(n_ref, x_ref, w_ref, b_ref, o_ref):
    @pl.when(pl.program_id(0) * BLK < n_ref[0])
    def _():
        o_ref[...] = _bdot(x_ref[...], w_ref[...]) + b_ref[...]


def _oproj(x, n, p):
    grid_spec = pltpu.PrefetchScalarGridSpec(
        num_scalar_prefetch=1,
        grid=(NBLK,),
        in_specs=[
            pl.BlockSpec((BLK, D), lambda i, n: (i, 0)),
            pl.BlockSpec((D, D), lambda i, n: (0, 0)),
            pl.BlockSpec((1, D), lambda i, n: (0, 0)),
        ],
        out_specs=pl.BlockSpec((BLK, D), lambda i, n: (i, 0)),
    )
    return pl.pallas_call(
        _oproj_kernel,
        grid_spec=grid_spec,
        out_shape=jax.ShapeDtypeStruct((S, D), jnp.float32),
    )(n, x, p['wo'].astype(jnp.bfloat16), p['bo'].reshape(1, D))


def _ffn_kernel(hs_ref, cnn_ref, attn_ref, s01_ref, s11_ref, s00_ref,
                w1_ref, b1_ref, w2_ref, b2_ref,
                g1_ref, lb1_ref, g2_ref, lb2_ref, out_ref):
    hs = hs_ref[...]
    combined = jnp.where(s00_ref[...] > 0.5, 0.0, hs)
    combined = jnp.where(s01_ref[...] > 0.5, cnn_ref[...], combined)
    combined = jnp.where(s11_ref[...] > 0.5, attn_ref[...], combined)
    x1 = _ln(hs + combined, g1_ref[...], lb1_ref[...])
    t = _gelu(_bdot(x1, w1_ref[...]) + b1_ref[...])
    f = _bdot(t, w2_ref[...]) + b2_ref[...]
    out_ref[...] = _ln(x1 + f, g2_ref[...], lb2_ref[...])


def _ffn(hs2d, cnn, attn, s01, s11, s00, p):
    return pl.pallas_call(
        _ffn_kernel,
        grid=(NBLK,),
        in_specs=[
            pl.BlockSpec((BLK, D), lambda i: (i, 0)),
            pl.BlockSpec((BLK, D), lambda i: (i, 0)),
            pl.BlockSpec((BLK, D), lambda i: (i, 0)),
            pl.BlockSpec((BLK, 1), lambda i: (i, 0)),
            pl.BlockSpec((BLK, 1), lambda i: (i, 0)),
            pl.BlockSpec((BLK, 1), lambda i: (i, 0)),
            pl.BlockSpec((D, FF), lambda i: (0, 0)),
            pl.BlockSpec((1, FF), lambda i: (0, 0)),
            pl.BlockSpec((FF, D), lambda i: (0, 0)),
            pl.BlockSpec((1, D), lambda i: (0, 0)),
            pl.BlockSpec((1, D), lambda i: (0, 0)),
            pl.BlockSpec((1, D), lambda i: (0, 0)),
            pl.BlockSpec((1, D), lambda i: (0, 0)),
            pl.BlockSpec((1, D), lambda i: (0, 0)),
        ],
        out_specs=pl.BlockSpec((BLK, D), lambda i: (i, 0)),
        out_shape=jax.ShapeDtypeStruct((S, D), jnp.float32),
    )(hs2d, cnn, attn, s01, s11, s00,
      p['ffn_w1'].astype(jnp.bfloat16), p['ffn_b1'].reshape(1, FF),
      p['ffn_w2'].astype(jnp.bfloat16), p['ffn_b2'].reshape(1, D),
      p['ln1_g'].reshape(1, D), p['ln1_b'].reshape(1, D),
      p['ln2_g'].reshape(1, D), p['ln2_b'].reshape(1, D))


def kernel(hidden_states, register_state, params):
    p = params
    hs2d = hidden_states.reshape(S, D)
    reg2d = register_state.reshape(1, D)

    cnn, cls, s01, s11, s00, m10stats = _router_cnn(hs2d, p)
    upd, kr, vr = _register(reg2d, m10stats, p)

    idxg, idxs, cnts = _compact(cls)
    n11 = cnts[0:1]

    x11 = _sc_gather(hs2d, idxg, cnts)
    q, k, v = _qkv(x11, n11, p)
    ao = _attention(q, k, v, kr, vr, n11)
    attn_cmp = _oproj(ao, n11, p)
    attn_pos = _sc_scatter(attn_cmp, idxs, cnts)

    out = _ffn(hs2d, cnn, attn_pos, s01, s11, s00, p)
    return out.reshape(1, S, D), upd.reshape(1, D)


# speedup vs baseline: 2.0001x; 1.2328x over previous
import dataclasses
import functools

import jax
import jax.numpy as jnp
import numpy as np
from jax.experimental import pallas as pl
from jax.experimental.pallas import tpu as pltpu
from jax.experimental.pallas import tpu_sc as plsc

D = 1024
H = 16
DH = 64
FF = 4096
KS = 5
S = 2048
BLK = 256
NBLK = S // BLK
INV_SCALE = 1.0 / np.sqrt(DH)

_NSUB = 32
_CHUNK = 16
_NCH = S // _CHUNK
_CPS = _NCH // _NSUB


def _gelu(x):
    return 0.5 * x * (1.0 + jax.lax.erf(x * np.float32(1.0 / np.sqrt(2.0))))


def _ln(x, g, b):
    m = x.mean(-1, keepdims=True)
    v = ((x - m) ** 2).mean(-1, keepdims=True)
    return (x - m) / jnp.sqrt(v + 1e-5) * g + b


def _dot(a, b):
    return jnp.dot(a, b, preferred_element_type=jnp.float32)


def _bdot(a, b):
    return jnp.dot(a.astype(jnp.bfloat16), b,
                   preferred_element_type=jnp.float32)


def _router_cnn_kernel(prev_ref, cur_ref, next_ref, rw1_ref, rb1_ref,
                       rw2_ref, rb2_ref, dww_ref, dwb_ref, pww_ref, pwb_ref,
                       cg_ref, cb_ref,
                       cnn_ref, cls_ref, s01_ref, s11_ref, s00_ref,
                       m10stats_ref):
    i = pl.program_id(0)
    x = cur_ref[...]
    h = _gelu(_dot(x, rw1_ref[...]) + rb1_ref[...])
    lg = _dot(h, rw2_ref[...]) + rb2_ref[...]
    mx = lg.max(axis=-1, keepdims=True)
    e = jnp.exp(lg - mx)
    conf0 = e[:, 0:1] / e.sum(axis=-1, keepdims=True)
    l0, l1, l2 = lg[:, 0:1], lg[:, 1:2], lg[:, 2:3]
    packets = jnp.where(l0 == mx, 0,
                        jnp.where(l1 == mx, 1,
                                  jnp.where(l2 == mx, 2, 3))).astype(jnp.int32)
    m00 = (packets == 0) & (conf0 > 0.99)
    cls = jnp.where(m00, 4, packets)
    cls_ref[0, 0, :] = cls.reshape(1, BLK)[0, :]
    s01_ref[...] = (packets == 1).astype(jnp.float32)
    s11_ref[...] = (packets == 3).astype(jnp.float32)
    s00_ref[...] = m00.astype(jnp.float32)
    m10 = (packets == 2).astype(jnp.float32)
    contrib = (x * m10).sum(axis=0, keepdims=True)
    ccnt = jnp.full((1, D), m10.sum(), jnp.float32)
    stats = jnp.concatenate([contrib, ccnt], axis=0)

    @pl.when(i == 0)
    def _():
        m10stats_ref[...] = stats

    @pl.when(i > 0)
    def _():
        m10stats_ref[...] += stats

    zero2 = jnp.zeros((2, D), jnp.float32)
    top = jnp.where(i > 0, prev_ref[BLK - 2:BLK, :], zero2)
    bot = jnp.where(i < NBLK - 1, next_ref[0:2, :], zero2)
    ext = jnp.concatenate([top, x, bot], axis=0)
    acc = dwb_ref[...]
    for j in range(KS):
        acc = acc + ext[j:j + BLK, :] * dww_ref[j:j + 1, :]
    conv = _gelu(acc)
    y = _bdot(conv, pww_ref[...]) + pwb_ref[...]
    cnn_ref[...] = _ln(y, cg_ref[...], cb_ref[...])


def _router_cnn(hs2d, p):
    dww = jnp.transpose(p['dw_w'][:, 0, :], (1, 0))
    cnn, cls3, s01, s11, s00, m10stats = pl.pallas_call(
        _router_cnn_kernel,
        grid=(NBLK,),
        in_specs=[
            pl.BlockSpec((BLK, D), lambda i: (jnp.maximum(i - 1, 0), 0)),
            pl.BlockSpec((BLK, D), lambda i: (i, 0)),
            pl.BlockSpec((BLK, D), lambda i: (jnp.minimum(i + 1, NBLK - 1), 0)),
            pl.BlockSpec((D, 64), lambda i: (0, 0)),
            pl.BlockSpec((1, 64), lambda i: (0, 0)),
            pl.BlockSpec((64, 4), lambda i: (0, 0)),
            pl.BlockSpec((1, 4), lambda i: (0, 0)),
            pl.BlockSpec((KS, D), lambda i: (0, 0)),
            pl.BlockSpec((1, D), lambda i: (0, 0)),
            pl.BlockSpec((D, D), lambda i: (0, 0)),
            pl.BlockSpec((1, D), lambda i: (0, 0)),
            pl.BlockSpec((1, D), lambda i: (0, 0)),
            pl.BlockSpec((1, D), lambda i: (0, 0)),
        ],
        out_specs=[
            pl.BlockSpec((BLK, D), lambda i: (i, 0)),
            pl.BlockSpec((1, 1, BLK), lambda i: (i, 0, 0)),
            pl.BlockSpec((BLK, 1), lambda i: (i, 0)),
            pl.BlockSpec((BLK, 1), lambda i: (i, 0)),
            pl.BlockSpec((BLK, 1), lambda i: (i, 0)),
            pl.BlockSpec((2, D), lambda i: (0, 0)),
        ],
        out_shape=[
            jax.ShapeDtypeStruct((S, D), jnp.float32),
            jax.ShapeDtypeStruct((NBLK, 1, BLK), jnp.int32),
            jax.ShapeDtypeStruct((S, 1), jnp.float32),
            jax.ShapeDtypeStruct((S, 1), jnp.float32),
            jax.ShapeDtypeStruct((S, 1), jnp.float32),
            jax.ShapeDtypeStruct((2, D), jnp.float32),
        ],
    )(hs2d, hs2d, hs2d, p['r_w1'], p['r_b1'].reshape(1, 64),
      p['r_w2'], p['r_b2'].reshape(1, 4), dww, p['dw_b'].reshape(1, D),
      p['pw_w'].astype(jnp.bfloat16), p['pw_b'].reshape(1, D),
      p['cnn_g'].reshape(1, D), p['cnn_b'].reshape(1, D))
    return cnn, cls3.reshape(S), s01, s11, s00, m10stats


def _register_kernel(reg_ref, m10stats_ref, alpha_ref,
                     regw_ref, regb_ref, reglng_ref, reglnb_ref,
                     wrk_ref, brk_ref, wrv_ref, brv_ref,
                     upd_ref, kr_ref, vr_ref):
    cnt = jnp.maximum(m10stats_ref[1:2, :], 1.0)
    anc_mean = _dot(m10stats_ref[0:1, :], regw_ref[...]) / cnt + regb_ref[...]
    reg = reg_ref[...]
    a = jax.nn.sigmoid(alpha_ref[...])
    upd_ref[...] = _ln(reg + a * (anc_mean - reg), reglng_ref[...], reglnb_ref[...])
    kr_ref[...] = _dot(reg, wrk_ref[...]) + brk_ref[...]
    vr_ref[...] = _dot(reg, wrv_ref[...]) + brv_ref[...]


def _register(reg2d, m10stats, p):
    full = lambda shp: pl.BlockSpec(shp, lambda: (0,) * len(shp))
    return pl.pallas_call(
        _register_kernel,
        in_specs=[full((1, D)), full((2, D)), full((1, 1)),
                  full((D, D)), full((1, D)), full((1, D)), full((1, D)),
                  full((D, D)), full((1, D)), full((D, D)), full((1, D))],
        out_specs=[full((1, D)), full((1, D)), full((1, D))],
        out_shape=[jax.ShapeDtypeStruct((1, D), jnp.float32)] * 3,
    )(reg2d, m10stats, p['alpha'].reshape(1, 1),
      p['reg_w'], p['reg_b'].reshape(1, D),
      p['regln_g'].reshape(1, D), p['regln_b'].reshape(1, D),
      p['wrk'], p['brk'].reshape(1, D), p['wrv'], p['brv'].reshape(1, D))


def _compact(cls):
    mesh = plsc.ScalarSubcoreMesh(axis_name='core', num_cores=1)

    @pl.kernel(
        out_type=[jax.ShapeDtypeStruct((S,), jnp.int32),
                  jax.ShapeDtypeStruct((S,), jnp.int32),
                  jax.ShapeDtypeStruct((16,), jnp.int32)],
        mesh=mesh,
        scratch_types=[pltpu.SMEM((S,), jnp.int32),
                       pltpu.SMEM((S,), jnp.int32),
                       pltpu.SMEM((S,), jnp.int32),
                       pltpu.SMEM((16,), jnp.int32),
                       pltpu.SemaphoreType.DMA],
    )
    def body(cls_hbm, idxg_hbm, idxs_hbm, cnt_hbm,
             cls_s, idxg_s, idxs_s, cnt_s, sem):
        pltpu.async_copy(cls_hbm, cls_s, sem).wait()

        @pl.loop(0, 16)
        def _(i):
            cnt_s[i] = 0

        @pl.loop(0, S)
        def _(i):
            idxg_s[i] = 0
            idxs_s[i] = S

            @pl.when(cls_s[i] == 3)
            def _():
                p = cnt_s[0]
                idxg_s[p] = i
                idxs_s[p] = i
                cnt_s[0] = p + 1

        pltpu.async_copy(idxg_s, idxg_hbm, sem).wait()
        pltpu.async_copy(idxs_s, idxs_hbm, sem).wait()
        pltpu.async_copy(cnt_s, cnt_hbm, sem).wait()

    idxg, idxs, cnts = body(cls)
    return idxg, idxs, cnts


def _sc_vec_params():
    cp = pltpu.CompilerParams()
    if "needs_layout_passes" in pltpu.CompilerParams.__dataclass_fields__:
        cp = dataclasses.replace(cp, needs_layout_passes=False)
    return cp


def _sc_gather(table, idx, cnts):
    mesh = plsc.VectorSubcoreMesh(core_axis_name='c', subcore_axis_name='s')

    @pl.kernel(
        out_type=jax.ShapeDtypeStruct((S, D), jnp.float32),
        mesh=mesh,
        compiler_params=_sc_vec_params(),
        scratch_types=[pltpu.VMEM((_CHUNK,), jnp.int32),
                       pltpu.VMEM((_CHUNK, D), jnp.float32),
                       pltpu.VMEM((16,), jnp.int32),
                       pltpu.SemaphoreType.DMA],
    )
    def body(table_hbm, idx_hbm, cnt_hbm, out_hbm, idx_v, rows_v, cnt_v, sem):
        pltpu.sync_copy(cnt_hbm, cnt_v)
        n = jnp.max(cnt_v[...])
        wid = jax.lax.axis_index('s') * 2 + jax.lax.axis_index('c')
        for c in range(_CPS):
            j = wid + c * _NSUB

            @pl.when(j * _CHUNK < n)
            def _(j=j):
                base = j * _CHUNK
                pltpu.sync_copy(idx_hbm.at[pl.ds(base, _CHUNK)], idx_v)
                pltpu.async_copy(table_hbm.at[idx_v], rows_v, sem).wait()
                pltpu.sync_copy(rows_v, out_hbm.at[pl.ds(base, _CHUNK)])

    return body(table, idx, cnts)


def _sc_scatter(rows, idx, cnts):
    mesh = plsc.VectorSubcoreMesh(core_axis_name='c', subcore_axis_name='s')

    @pl.kernel(
        out_type=jax.ShapeDtypeStruct((S + _CHUNK, D), jnp.float32),
        mesh=mesh,
        compiler_params=_sc_vec_params(),
        scratch_types=[pltpu.VMEM((_CHUNK,), jnp.int32),
                       pltpu.VMEM((_CHUNK, D), jnp.float32),
                       pltpu.VMEM((16,), jnp.int32),
                       pltpu.SemaphoreType.DMA],
    )
    def body(rows_hbm, idx_hbm, cnt_hbm, out_hbm, idx_v, rows_v, cnt_v, sem):
        pltpu.sync_copy(cnt_hbm, cnt_v)
        n = jnp.max(cnt_v[...])
        wid = jax.lax.axis_index('s') * 2 + jax.lax.axis_index('c')
        for c in range(_CPS):
            j = wid + c * _NSUB

            @pl.when(j * _CHUNK < n)
            def _(j=j):
                base = j * _CHUNK
                pltpu.sync_copy(idx_hbm.at[pl.ds(base, _CHUNK)], idx_v)
                pltpu.sync_copy(rows_hbm.at[pl.ds(base, _CHUNK)], rows_v)
                pltpu.async_copy(rows_v, out_hbm.at[idx_v], sem).wait()

    return body(rows, idx, cnts)


def _qkv_kernel(n_ref, x_ref, wq_ref, bq_ref, wk_ref, bk_ref, wv_ref, bv_ref,
                q_ref, k_ref, v_ref):
    i = pl.program_id(0)

    @pl.when(i * BLK < n_ref[0])
    def _():
        ridx = i * BLK + jax.lax.broadcasted_iota(jnp.int32, (BLK, 1), 0)
        x = jnp.where(ridx < n_ref[0], x_ref[...], 0.0).astype(jnp.bfloat16)
        q_ref[...] = _dot(x, wq_ref[...]) + bq_ref[...]
        k_ref[...] = _dot(x, wk_ref[...]) + bk_ref[...]
        v_ref[...] = _dot(x, wv_ref[...]) + bv_ref[...]


def _qkv(x11, n11, p):
    grid_spec = pltpu.PrefetchScalarGridSpec(
        num_scalar_prefetch=1,
        grid=(NBLK,),
        in_specs=[
            pl.BlockSpec((BLK, D), lambda i, n: (i, 0)),
            pl.BlockSpec((D, D), lambda i, n: (0, 0)),
            pl.BlockSpec((1, D), lambda i, n: (0, 0)),
            pl.BlockSpec((D, D), lambda i, n: (0, 0)),
            pl.BlockSpec((1, D), lambda i, n: (0, 0)),
            pl.BlockSpec((D, D), lambda i, n: (0, 0)),
            pl.BlockSpec((1, D), lambda i, n: (0, 0)),
        ],
        out_specs=[pl.BlockSpec((BLK, D), lambda i, n: (i, 0))] * 3,
    )
    return pl.pallas_call(
        _qkv_kernel,
        grid_spec=grid_spec,
        out_shape=[jax.ShapeDtypeStruct((S, D), jnp.float32)] * 3,
    )(n11, x11, p['wq'].astype(jnp.bfloat16), p['bq'].reshape(1, D),
      p['wk'].astype(jnp.bfloat16), p['bk'].reshape(1, D),
      p['wv'].astype(jnp.bfloat16), p['bv'].reshape(1, D))


def _attn_kernel(n_ref, q_ref, k_ref, v_ref, kr_ref, vr_ref, ao_ref):
    qi = pl.program_id(1)
    n = n_ref[0]
    nblk = (n + BLK - 1) // BLK

    @pl.when(qi * BLK < n)
    def _():
        for t in range(1, NBLK + 1):
            @pl.when(nblk == t)
            def _(t=t):
                outs = []
                for off in (0, DH):
                    q = q_ref[:, off:off + DH]
                    qb = q.astype(jnp.bfloat16)
                    sreg = (q * kr_ref[:, off:off + DH]).sum(
                        axis=-1, keepdims=True) * INV_SCALE
                    kt = k_ref[0:t * BLK, off:off + DH]
                    s = _dot(qb, kt.astype(jnp.bfloat16).T) * INV_SCALE
                    kidx = jax.lax.broadcasted_iota(
                        jnp.int32, (BLK, t * BLK), 1)
                    s = jnp.where(kidx < n, s, -1e9)
                    m = jnp.maximum(s.max(axis=-1, keepdims=True), sreg)
                    w = jnp.exp(s - m)
                    wr = jnp.exp(sreg - m)
                    den = w.sum(axis=-1, keepdims=True) + wr
                    vt = v_ref[0:t * BLK, off:off + DH]
                    o = _dot(w.astype(jnp.bfloat16), vt.astype(jnp.bfloat16))
                    outs.append((o + wr * vr_ref[:, off:off + DH]) / den)
                ao_ref[...] = jnp.concatenate(outs, axis=1)


def _attention(q, k, v, kr, vr, n11):
    grid_spec = pltpu.PrefetchScalarGridSpec(
        num_scalar_prefetch=1,
        grid=(H // 2, NBLK),
        in_specs=[
            pl.BlockSpec((BLK, 2 * DH), lambda h, qi, n: (qi, h)),
            pl.BlockSpec((S, 2 * DH), lambda h, qi, n: (0, h)),
            pl.BlockSpec((S, 2 * DH), lambda h, qi, n: (0, h)),
            pl.BlockSpec((1, 2 * DH), lambda h, qi, n: (0, h)),
            pl.BlockSpec((1, 2 * DH), lambda h, qi, n: (0, h)),
        ],
        out_specs=pl.BlockSpec((BLK, 2 * DH), lambda h, qi, n: (qi, h)),
    )
    return pl.pallas_call(
        _attn_kernel,
        grid_spec=grid_spec,
        out_shape=jax.ShapeDtypeStruct((S, D), jnp.float32),
    )(n11, q, k, v, kr, vr)


def _oproj_kernel(n_ref, x_ref, w_ref, b_ref, o_ref):
    @pl.when(pl.program_id(0) * BLK < n_ref[0])
    def _():
        o_ref[...] = _bdot(x_ref[...], w_ref[...]) + b_ref[...]


def _oproj(x, n, p):
    grid_spec = pltpu.PrefetchScalarGridSpec(
        num_scalar_prefetch=1,
        grid=(NBLK,),
        in_specs=[
            pl.BlockSpec((BLK, D), lambda i, n: (i, 0)),
            pl.BlockSpec((D, D), lambda i, n: (0, 0)),
            pl.BlockSpec((1, D), lambda i, n: (0, 0)),
        ],
        out_specs=pl.BlockSpec((BLK, D), lambda i, n: (i, 0)),
    )
    return pl.pallas_call(
        _oproj_kernel,
        grid_spec=grid_spec,
        out_shape=jax.ShapeDtypeStruct((S, D), jnp.float32),
    )(n, x, p['wo'].astype(jnp.bfloat16), p['bo'].reshape(1, D))


def _ffn_kernel(hs_ref, cnn_ref, attn_ref, s01_ref, s11_ref, s00_ref,
                w1_ref, b1_ref, w2_ref, b2_ref,
                g1_ref, lb1_ref, g2_ref, lb2_ref, out_ref):
    hs = hs_ref[...]
    combined = jnp.where(s00_ref[...] > 0.5, 0.0, hs)
    combined = jnp.where(s01_ref[...] > 0.5, cnn_ref[...], combined)
    combined = jnp.where(s11_ref[...] > 0.5, attn_ref[...], combined)
    x1 = _ln(hs + combined, g1_ref[...], lb1_ref[...])
    t = _gelu(_bdot(x1, w1_ref[...]) + b1_ref[...])
    f = _bdot(t, w2_ref[...]) + b2_ref[...]
    out_ref[...] = _ln(x1 + f, g2_ref[...], lb2_ref[...])


def _ffn(hs2d, cnn, attn, s01, s11, s00, p):
    return pl.pallas_call(
        _ffn_kernel,
        grid=(NBLK,),
        in_specs=[
            pl.BlockSpec((BLK, D), lambda i: (i, 0)),
            pl.BlockSpec((BLK, D), lambda i: (i, 0)),
            pl.BlockSpec((BLK, D), lambda i: (i, 0)),
            pl.BlockSpec((BLK, 1), lambda i: (i, 0)),
            pl.BlockSpec((BLK, 1), lambda i: (i, 0)),
            pl.BlockSpec((BLK, 1), lambda i: (i, 0)),
            pl.BlockSpec((D, FF), lambda i: (0, 0)),
            pl.BlockSpec((1, FF), lambda i: (0, 0)),
            pl.BlockSpec((FF, D), lambda i: (0, 0)),
            pl.BlockSpec((1, D), lambda i: (0, 0)),
            pl.BlockSpec((1, D), lambda i: (0, 0)),
            pl.BlockSpec((1, D), lambda i: (0, 0)),
            pl.BlockSpec((1, D), lambda i: (0, 0)),
            pl.BlockSpec((1, D), lambda i: (0, 0)),
        ],
        out_specs=pl.BlockSpec((BLK, D), lambda i: (i, 0)),
        out_shape=jax.ShapeDtypeStruct((S, D), jnp.float32),
    )(hs2d, cnn, attn, s01, s11, s00,
      p['ffn_w1'].astype(jnp.bfloat16), p['ffn_b1'].reshape(1, FF),
      p['ffn_w2'].astype(jnp.bfloat16), p['ffn_b2'].reshape(1, D),
      p['ln1_g'].reshape(1, D), p['ln1_b'].reshape(1, D),
      p['ln2_g'].reshape(1, D), p['ln2_b'].reshape(1, D))


def kernel(hidden_states, register_state, params):
    p = params
    hs2d = hidden_states.reshape(S, D)
    reg2d = register_state.reshape(1, D)

    cnn, cls, s01, s11, s00, m10stats = _router_cnn(hs2d, p)
    upd, kr, vr = _register(reg2d, m10stats, p)

    idxg, idxs, cnts = _compact(cls)
    n11 = cnts[0:1]

    x11 = _sc_gather(hs2d, idxg, cnts)
    q, k, v = _qkv(x11, n11, p)
    ao = _attention(q, k, v, kr, vr, n11)
    attn_cmp = _oproj(ao, n11, p)
    attn_pos = _sc_scatter(attn_cmp, idxs, cnts)

    out = _ffn(hs2d, cnn, attn_pos, s01, s11, s00, p)
    return out.reshape(1, S, D), upd.reshape(1, D)


# speedup vs baseline: 2.0460x; 1.0230x over previous
import dataclasses
import functools

import jax
import jax.numpy as jnp
import numpy as np
from jax.experimental import pallas as pl
from jax.experimental.pallas import tpu as pltpu
from jax.experimental.pallas import tpu_sc as plsc

D = 1024
H = 16
DH = 64
FF = 4096
KS = 5
S = 2048
BLK = 256
NBLK = S // BLK
INV_SCALE = 1.0 / np.sqrt(DH)

_NSUB = 32
_CHUNK = 16
_NCH = S // _CHUNK
_CPS = _NCH // _NSUB


def _gelu(x):
    return 0.5 * x * (1.0 + jax.lax.erf(x * np.float32(1.0 / np.sqrt(2.0))))


def _ln(x, g, b):
    m = x.mean(-1, keepdims=True)
    v = ((x - m) ** 2).mean(-1, keepdims=True)
    return (x - m) / jnp.sqrt(v + 1e-5) * g + b


def _dot(a, b):
    return jnp.dot(a, b, preferred_element_type=jnp.float32)


def _bdot(a, b):
    return jnp.dot(a.astype(jnp.bfloat16), b,
                   preferred_element_type=jnp.float32)


def _router_kernel(x_ref, rw1_ref, rb1_ref, rw2_ref, rb2_ref,
                   cls_ref, s01_ref, s11_ref, s00_ref, m10stats_ref):
    i = pl.program_id(0)
    x = x_ref[...]
    h = _gelu(_dot(x, rw1_ref[...]) + rb1_ref[...])
    lg = _dot(h, rw2_ref[...]) + rb2_ref[...]
    mx = lg.max(axis=-1, keepdims=True)
    e = jnp.exp(lg - mx)
    conf0 = e[:, 0:1] / e.sum(axis=-1, keepdims=True)
    l0, l1, l2 = lg[:, 0:1], lg[:, 1:2], lg[:, 2:3]
    packets = jnp.where(l0 == mx, 0,
                        jnp.where(l1 == mx, 1,
                                  jnp.where(l2 == mx, 2, 3))).astype(jnp.int32)
    m00 = (packets == 0) & (conf0 > 0.99)
    cls = jnp.where(m00, 4, packets)
    cls_ref[0, 0, :] = cls.reshape(1, BLK)[0, :]
    s01_ref[...] = (packets == 1).astype(jnp.float32)
    s11_ref[...] = (packets == 3).astype(jnp.float32)
    s00_ref[...] = m00.astype(jnp.float32)
    m10 = (packets == 2).astype(jnp.float32)
    contrib = (x * m10).sum(axis=0, keepdims=True)
    ccnt = jnp.full((1, D), m10.sum(), jnp.float32)
    stats = jnp.concatenate([contrib, ccnt], axis=0)

    @pl.when(i == 0)
    def _():
        m10stats_ref[...] = stats

    @pl.when(i > 0)
    def _():
        m10stats_ref[...] += stats


def _router(hs2d, p):
    cls3, s01, s11, s00, m10stats = pl.pallas_call(
        _router_kernel,
        grid=(NBLK,),
        in_specs=[
            pl.BlockSpec((BLK, D), lambda i: (i, 0)),
            pl.BlockSpec((D, 64), lambda i: (0, 0)),
            pl.BlockSpec((1, 64), lambda i: (0, 0)),
            pl.BlockSpec((64, 4), lambda i: (0, 0)),
            pl.BlockSpec((1, 4), lambda i: (0, 0)),
        ],
        out_specs=[
            pl.BlockSpec((1, 1, BLK), lambda i: (i, 0, 0)),
            pl.BlockSpec((BLK, 1), lambda i: (i, 0)),
            pl.BlockSpec((BLK, 1), lambda i: (i, 0)),
            pl.BlockSpec((BLK, 1), lambda i: (i, 0)),
            pl.BlockSpec((2, D), lambda i: (0, 0)),
        ],
        out_shape=[
            jax.ShapeDtypeStruct((NBLK, 1, BLK), jnp.int32),
            jax.ShapeDtypeStruct((S, 1), jnp.float32),
            jax.ShapeDtypeStruct((S, 1), jnp.float32),
            jax.ShapeDtypeStruct((S, 1), jnp.float32),
            jax.ShapeDtypeStruct((2, D), jnp.float32),
        ],
    )(hs2d, p['r_w1'], p['r_b1'].reshape(1, 64),
      p['r_w2'], p['r_b2'].reshape(1, 4))
    return cls3.reshape(S), s01, s11, s00, m10stats


def _cnn_kernel(prev_ref, cur_ref, next_ref, dww_ref, dwb_ref,
                pww_ref, pwb_ref, cg_ref, cb_ref, cnn_ref):
    i = pl.program_id(0)
    x = cur_ref[...]
    zero2 = jnp.zeros((2, D), jnp.float32)
    top = jnp.where(i > 0, prev_ref[BLK - 2:BLK, :], zero2)
    bot = jnp.where(i < NBLK - 1, next_ref[0:2, :], zero2)
    ext = jnp.concatenate([top, x, bot], axis=0)
    acc = dwb_ref[...]
    for j in range(KS):
        acc = acc + ext[j:j + BLK, :] * dww_ref[j:j + 1, :]
    conv = _gelu(acc)
    y = _bdot(conv, pww_ref[...]) + pwb_ref[...]
    cnn_ref[...] = _ln(y, cg_ref[...], cb_ref[...])


def _cnn(hs2d, p):
    dww = jnp.transpose(p['dw_w'][:, 0, :], (1, 0))
    return pl.pallas_call(
        _cnn_kernel,
        grid=(NBLK,),
        in_specs=[
            pl.BlockSpec((BLK, D), lambda i: (jnp.maximum(i - 1, 0), 0)),
            pl.BlockSpec((BLK, D), lambda i: (i, 0)),
            pl.BlockSpec((BLK, D), lambda i: (jnp.minimum(i + 1, NBLK - 1), 0)),
            pl.BlockSpec((KS, D), lambda i: (0, 0)),
            pl.BlockSpec((1, D), lambda i: (0, 0)),
            pl.BlockSpec((D, D), lambda i: (0, 0)),
            pl.BlockSpec((1, D), lambda i: (0, 0)),
            pl.BlockSpec((1, D), lambda i: (0, 0)),
            pl.BlockSpec((1, D), lambda i: (0, 0)),
        ],
        out_specs=pl.BlockSpec((BLK, D), lambda i: (i, 0)),
        out_shape=jax.ShapeDtypeStruct((S, D), jnp.float32),
    )(hs2d, hs2d, hs2d, dww, p['dw_b'].reshape(1, D),
      p['pw_w'].astype(jnp.bfloat16), p['pw_b'].reshape(1, D),
      p['cnn_g'].reshape(1, D), p['cnn_b'].reshape(1, D))


def _register_kernel(reg_ref, m10stats_ref, alpha_ref,
                     regw_ref, regb_ref, reglng_ref, reglnb_ref,
                     wrk_ref, brk_ref, wrv_ref, brv_ref,
                     upd_ref, kr_ref, vr_ref):
    cnt = jnp.maximum(m10stats_ref[1:2, :], 1.0)
    anc_mean = _dot(m10stats_ref[0:1, :], regw_ref[...]) / cnt + regb_ref[...]
    reg = reg_ref[...]
    a = jax.nn.sigmoid(alpha_ref[...])
    upd_ref[...] = _ln(reg + a * (anc_mean - reg), reglng_ref[...], reglnb_ref[...])
    kr_ref[...] = _dot(reg, wrk_ref[...]) + brk_ref[...]
    vr_ref[...] = _dot(reg, wrv_ref[...]) + brv_ref[...]


def _register(reg2d, m10stats, p):
    full = lambda shp: pl.BlockSpec(shp, lambda: (0,) * len(shp))
    return pl.pallas_call(
        _register_kernel,
        in_specs=[full((1, D)), full((2, D)), full((1, 1)),
                  full((D, D)), full((1, D)), full((1, D)), full((1, D)),
                  full((D, D)), full((1, D)), full((D, D)), full((1, D))],
        out_specs=[full((1, D)), full((1, D)), full((1, D))],
        out_shape=[jax.ShapeDtypeStruct((1, D), jnp.float32)] * 3,
    )(reg2d, m10stats, p['alpha'].reshape(1, 1),
      p['reg_w'], p['reg_b'].reshape(1, D),
      p['regln_g'].reshape(1, D), p['regln_b'].reshape(1, D),
      p['wrk'], p['brk'].reshape(1, D), p['wrv'], p['brv'].reshape(1, D))


def _compact(cls):
    mesh = plsc.ScalarSubcoreMesh(axis_name='core', num_cores=1)

    @pl.kernel(
        out_type=[jax.ShapeDtypeStruct((S,), jnp.int32),
                  jax.ShapeDtypeStruct((S,), jnp.int32),
                  jax.ShapeDtypeStruct((16,), jnp.int32)],
        mesh=mesh,
        scratch_types=[pltpu.SMEM((S,), jnp.int32),
                       pltpu.SMEM((S,), jnp.int32),
                       pltpu.SMEM((S,), jnp.int32),
                       pltpu.SMEM((16,), jnp.int32),
                       pltpu.SemaphoreType.DMA],
    )
    def body(cls_hbm, idxg_hbm, idxs_hbm, cnt_hbm,
             cls_s, idxg_s, idxs_s, cnt_s, sem):
        pltpu.async_copy(cls_hbm, cls_s, sem).wait()

        @pl.loop(0, 16)
        def _(i):
            cnt_s[i] = 0

        @pl.loop(0, S)
        def _(i):
            @pl.when(cls_s[i] == 3)
            def _():
                p = cnt_s[0]
                idxg_s[p] = i
                idxs_s[p] = i
                cnt_s[0] = p + 1

        n = cnt_s[0]
        hi = jnp.minimum(((n + _CHUNK - 1) // _CHUNK) * _CHUNK, S)

        @pl.loop(n, hi)
        def _(i):
            idxg_s[i] = 0
            idxs_s[i] = S

        pltpu.async_copy(idxg_s, idxg_hbm, sem).wait()
        pltpu.async_copy(idxs_s, idxs_hbm, sem).wait()
        pltpu.async_copy(cnt_s, cnt_hbm, sem).wait()

    idxg, idxs, cnts = body(cls)
    return idxg, idxs, cnts


def _sc_vec_params():
    cp = pltpu.CompilerParams()
    if "needs_layout_passes" in pltpu.CompilerParams.__dataclass_fields__:
        cp = dataclasses.replace(cp, needs_layout_passes=False)
    return cp


def _sc_gather(table, idx, cnts):
    mesh = plsc.VectorSubcoreMesh(core_axis_name='c', subcore_axis_name='s')

    @pl.kernel(
        out_type=jax.ShapeDtypeStruct((S, D), jnp.float32),
        mesh=mesh,
        compiler_params=_sc_vec_params(),
        scratch_types=[pltpu.VMEM((_CHUNK,), jnp.int32),
                       pltpu.VMEM((_CHUNK, D), jnp.float32),
                       pltpu.VMEM((16,), jnp.int32),
                       pltpu.SemaphoreType.DMA],
    )
    def body(table_hbm, idx_hbm, cnt_hbm, out_hbm, idx_v, rows_v, cnt_v, sem):
        pltpu.sync_copy(cnt_hbm, cnt_v)
        n = jnp.max(cnt_v[...])
        wid = jax.lax.axis_index('s') * 2 + jax.lax.axis_index('c')
        for c in range(_CPS):
            j = wid + c * _NSUB

            @pl.when(j * _CHUNK < n)
            def _(j=j):
                base = j * _CHUNK
                pltpu.sync_copy(idx_hbm.at[pl.ds(base, _CHUNK)], idx_v)
                pltpu.async_copy(table_hbm.at[idx_v], rows_v, sem).wait()
                pltpu.sync_copy(rows_v, out_hbm.at[pl.ds(base, _CHUNK)])

    return body(table, idx, cnts)


def _sc_scatter(rows, idx, cnts):
    mesh = plsc.VectorSubcoreMesh(core_axis_name='c', subcore_axis_name='s')

    @pl.kernel(
        out_type=jax.ShapeDtypeStruct((S + _CHUNK, D), jnp.float32),
        mesh=mesh,
        compiler_params=_sc_vec_params(),
        scratch_types=[pltpu.VMEM((_CHUNK,), jnp.int32),
                       pltpu.VMEM((_CHUNK, D), jnp.float32),
                       pltpu.VMEM((16,), jnp.int32),
                       pltpu.SemaphoreType.DMA],
    )
    def body(rows_hbm, idx_hbm, cnt_hbm, out_hbm, idx_v, rows_v, cnt_v, sem):
        pltpu.sync_copy(cnt_hbm, cnt_v)
        n = jnp.max(cnt_v[...])
        wid = jax.lax.axis_index('s') * 2 + jax.lax.axis_index('c')
        for c in range(_CPS):
            j = wid + c * _NSUB

            @pl.when(j * _CHUNK < n)
            def _(j=j):
                base = j * _CHUNK
                pltpu.sync_copy(idx_hbm.at[pl.ds(base, _CHUNK)], idx_v)
                pltpu.sync_copy(rows_hbm.at[pl.ds(base, _CHUNK)], rows_v)
                pltpu.async_copy(rows_v, out_hbm.at[idx_v], sem).wait()

    return body(rows, idx, cnts)


def _qkv_kernel(n_ref, x_ref, wq_ref, bq_ref, wk_ref, bk_ref, wv_ref, bv_ref,
                q_ref, k_ref, v_ref):
    i = pl.program_id(0)

    @pl.when(i * BLK < n_ref[0])
    def _():
        ridx = i * BLK + jax.lax.broadcasted_iota(jnp.int32, (BLK, 1), 0)
        x = jnp.where(ridx < n_ref[0], x_ref[...], 0.0).astype(jnp.bfloat16)
        q_ref[...] = _dot(x, wq_ref[...]) + bq_ref[...]
        k_ref[...] = _dot(x, wk_ref[...]) + bk_ref[...]
        v_ref[...] = _dot(x, wv_ref[...]) + bv_ref[...]


def _qkv(x11, n11, p):
    grid_spec = pltpu.PrefetchScalarGridSpec(
        num_scalar_prefetch=1,
        grid=(NBLK,),
        in_specs=[
            pl.BlockSpec((BLK, D), lambda i, n: (i, 0)),
            pl.BlockSpec((D, D), lambda i, n: (0, 0)),
            pl.BlockSpec((1, D), lambda i, n: (0, 0)),
            pl.BlockSpec((D, D), lambda i, n: (0, 0)),
            pl.BlockSpec((1, D), lambda i, n: (0, 0)),
            pl.BlockSpec((D, D), lambda i, n: (0, 0)),
            pl.BlockSpec((1, D), lambda i, n: (0, 0)),
        ],
        out_specs=[pl.BlockSpec((BLK, D), lambda i, n: (i, 0))] * 3,
    )
    return pl.pallas_call(
        _qkv_kernel,
        grid_spec=grid_spec,
        out_shape=[jax.ShapeDtypeStruct((S, D), jnp.float32)] * 3,
    )(n11, x11, p['wq'].astype(jnp.bfloat16), p['bq'].reshape(1, D),
      p['wk'].astype(jnp.bfloat16), p['bk'].reshape(1, D),
      p['wv'].astype(jnp.bfloat16), p['bv'].reshape(1, D))


def _attn_kernel(n_ref, q_ref, k_ref, v_ref, kr_ref, vr_ref, ao_ref):
    qi = pl.program_id(1)
    n = n_ref[0]
    nblk = (n + BLK - 1) // BLK

    @pl.when(qi * BLK < n)
    def _():
        for t in range(1, NBLK + 1):
            @pl.when(nblk == t)
            def _(t=t):
                outs = []
                for off in (0, DH):
                    q = q_ref[:, off:off + DH]
                    qb = q.astype(jnp.bfloat16)
                    sreg = (q * kr_ref[:, off:off + DH]).sum(
                        axis=-1, keepdims=True) * INV_SCALE
                    kt = k_ref[0:t * BLK, off:off + DH]
                    s = _dot(qb, kt.astype(jnp.bfloat16).T) * INV_SCALE
                    kidx = jax.lax.broadcasted_iota(
                        jnp.int32, (BLK, t * BLK), 1)
                    s = jnp.where(kidx < n, s, -1e9)
                    m = jnp.maximum(s.max(axis=-1, keepdims=True), sreg)
                    w = jnp.exp(s - m)
                    wr = jnp.exp(sreg - m)
                    den = w.sum(axis=-1, keepdims=True) + wr
                    vt = v_ref[0:t * BLK, off:off + DH]
                    o = _dot(w.astype(jnp.bfloat16), vt.astype(jnp.bfloat16))
                    outs.append((o + wr * vr_ref[:, off:off + DH]) / den)
                ao_ref[...] = jnp.concatenate(outs, axis=1)


def _attention(q, k, v, kr, vr, n11):
    grid_spec = pltpu.PrefetchScalarGridSpec(
        num_scalar_prefetch=1,
        grid=(H // 2, NBLK),
        in_specs=[
            pl.BlockSpec((BLK, 2 * DH), lambda h, qi, n: (qi, h)),
            pl.BlockSpec((S, 2 * DH), lambda h, qi, n: (0, h)),
            pl.BlockSpec((S, 2 * DH), lambda h, qi, n: (0, h)),
            pl.BlockSpec((1, 2 * DH), lambda h, qi, n: (0, h)),
            pl.BlockSpec((1, 2 * DH), lambda h, qi, n: (0, h)),
        ],
        out_specs=pl.BlockSpec((BLK, 2 * DH), lambda h, qi, n: (qi, h)),
    )
    return pl.pallas_call(
        _attn_kernel,
        grid_spec=grid_spec,
        out_shape=jax.ShapeDtypeStruct((S, D), jnp.float32),
    )(n11, q, k, v, kr, vr)


def _oproj_kernel(n_ref, x_ref, w_ref, b_ref, o_ref):
    @pl.when(pl.program_id(0) * BLK < n_ref[0])
    def _():
        o_ref[...] = _bdot(x_ref[...], w_ref[...]) + b_ref[...]


def _oproj(x, n, p):
    grid_spec = pltpu.PrefetchScalarGridSpec(
        num_scalar_prefetch=1,
        grid=(NBLK,),
        in_specs=[
            pl.BlockSpec((BLK, D), lambda i, n: (i, 0)),
            pl.BlockSpec((D, D), lambda i, n: (0, 0)),
            pl.BlockSpec((1, D), lambda i, n: (0, 0)),
        ],
        out_specs=pl.BlockSpec((BLK, D), lambda i, n: (i, 0)),
    )
    return pl.pallas_call(
        _oproj_kernel,
        grid_spec=grid_spec,
        out_shape=jax.ShapeDtypeStruct((S, D), jnp.float32),
    )(n, x, p['wo'].astype(jnp.bfloat16), p['bo'].reshape(1, D))


def _ffn_kernel(hs_ref, cnn_ref, attn_ref, s01_ref, s11_ref, s00_ref,
                w1_ref, b1_ref, w2_ref, b2_ref,
                g1_ref, lb1_ref, g2_ref, lb2_ref, out_ref):
    hs = hs_ref[...]
    combined = jnp.where(s00_ref[...] > 0.5, 0.0, hs)
    combined = jnp.where(s01_ref[...] > 0.5, cnn_ref[...], combined)
    combined = jnp.where(s11_ref[...] > 0.5, attn_ref[...], combined)
    x1 = _ln(hs + combined, g1_ref[...], lb1_ref[...])
    t = _gelu(_bdot(x1, w1_ref[...]) + b1_ref[...])
    f = _bdot(t, w2_ref[...]) + b2_ref[...]
    out_ref[...] = _ln(x1 + f, g2_ref[...], lb2_ref[...])


def _ffn(hs2d, cnn, attn, s01, s11, s00, p):
    return pl.pallas_call(
        _ffn_kernel,
        grid=(NBLK,),
        in_specs=[
            pl.BlockSpec((BLK, D), lambda i: (i, 0)),
            pl.BlockSpec((BLK, D), lambda i: (i, 0)),
            pl.BlockSpec((BLK, D), lambda i: (i, 0)),
            pl.BlockSpec((BLK, 1), lambda i: (i, 0)),
            pl.BlockSpec((BLK, 1), lambda i: (i, 0)),
            pl.BlockSpec((BLK, 1), lambda i: (i, 0)),
            pl.BlockSpec((D, FF), lambda i: (0, 0)),
            pl.BlockSpec((1, FF), lambda i: (0, 0)),
            pl.BlockSpec((FF, D), lambda i: (0, 0)),
            pl.BlockSpec((1, D), lambda i: (0, 0)),
            pl.BlockSpec((1, D), lambda i: (0, 0)),
            pl.BlockSpec((1, D), lambda i: (0, 0)),
            pl.BlockSpec((1, D), lambda i: (0, 0)),
            pl.BlockSpec((1, D), lambda i: (0, 0)),
        ],
        out_specs=pl.BlockSpec((BLK, D), lambda i: (i, 0)),
        out_shape=jax.ShapeDtypeStruct((S, D), jnp.float32),
    )(hs2d, cnn, attn, s01, s11, s00,
      p['ffn_w1'].astype(jnp.bfloat16), p['ffn_b1'].reshape(1, FF),
      p['ffn_w2'].astype(jnp.bfloat16), p['ffn_b2'].reshape(1, D),
      p['ln1_g'].reshape(1, D), p['ln1_b'].reshape(1, D),
      p['ln2_g'].reshape(1, D), p['ln2_b'].reshape(1, D))


def kernel(hidden_states, register_state, params):
    p = params
    hs2d = hidden_states.reshape(S, D)
    reg2d = register_state.reshape(1, D)

    cls, s01, s11, s00, m10stats = _router(hs2d, p)
    idxg, idxs, cnts = _compact(cls)
    cnn = _cnn(hs2d, p)
    upd, kr, vr = _register(reg2d, m10stats, p)
    n11 = cnts[0:1]

    x11 = _sc_gather(hs2d, idxg, cnts)
    q, k, v = _qkv(x11, n11, p)
    ao = _attention(q, k, v, kr, vr, n11)
    attn_cmp = _oproj(ao, n11, p)
    attn_pos = _sc_scatter(attn_cmp, idxs, cnts)

    out = _ffn(hs2d, cnn, attn_pos, s01, s11, s00, p)
    return out.reshape(1, S, D), upd.reshape(1, D)


# speedup vs baseline: 2.0512x; 1.0025x over previous
import dataclasses
import functools

import jax
import jax.numpy as jnp
import numpy as np
from jax.experimental import pallas as pl
from jax.experimental.pallas import tpu as pltpu
from jax.experimental.pallas import tpu_sc as plsc

D = 1024
H = 16
DH = 64
FF = 4096
KS = 5
S = 2048
BLK = 256
NBLK = S // BLK
INV_SCALE = 1.0 / np.sqrt(DH)

_NSUB = 32
_CHUNK = 32
_NCH = S // _CHUNK
_CPS = _NCH // _NSUB


def _gelu(x):
    return 0.5 * x * (1.0 + jax.lax.erf(x * np.float32(1.0 / np.sqrt(2.0))))


def _ln(x, g, b):
    m = x.mean(-1, keepdims=True)
    v = ((x - m) ** 2).mean(-1, keepdims=True)
    return (x - m) / jnp.sqrt(v + 1e-5) * g + b


def _dot(a, b):
    return jnp.dot(a, b, preferred_element_type=jnp.float32)


def _bdot(a, b):
    return jnp.dot(a.astype(jnp.bfloat16), b,
                   preferred_element_type=jnp.float32)


def _router_kernel(x_ref, rw1_ref, rb1_ref, rw2_ref, rb2_ref,
                   cls_ref, s01_ref, s11_ref, s00_ref, m10stats_ref):
    i = pl.program_id(0)
    x = x_ref[...]
    h = _gelu(_dot(x, rw1_ref[...]) + rb1_ref[...])
    lg = _dot(h, rw2_ref[...]) + rb2_ref[...]
    mx = lg.max(axis=-1, keepdims=True)
    e = jnp.exp(lg - mx)
    conf0 = e[:, 0:1] / e.sum(axis=-1, keepdims=True)
    l0, l1, l2 = lg[:, 0:1], lg[:, 1:2], lg[:, 2:3]
    packets = jnp.where(l0 == mx, 0,
                        jnp.where(l1 == mx, 1,
                                  jnp.where(l2 == mx, 2, 3))).astype(jnp.int32)
    m00 = (packets == 0) & (conf0 > 0.99)
    cls = jnp.where(m00, 4, packets)
    cls_ref[0, 0, :] = cls.reshape(1, BLK)[0, :]
    s01_ref[...] = (packets == 1).astype(jnp.float32)
    s11_ref[...] = (packets == 3).astype(jnp.float32)
    s00_ref[...] = m00.astype(jnp.float32)
    m10 = (packets == 2).astype(jnp.float32)
    contrib = (x * m10).sum(axis=0, keepdims=True)
    ccnt = jnp.full((1, D), m10.sum(), jnp.float32)
    stats = jnp.concatenate([contrib, ccnt], axis=0)

    @pl.when(i == 0)
    def _():
        m10stats_ref[...] = stats

    @pl.when(i > 0)
    def _():
        m10stats_ref[...] += stats


def _router(hs2d, p):
    cls3, s01, s11, s00, m10stats = pl.pallas_call(
        _router_kernel,
        grid=(NBLK,),
        in_specs=[
            pl.BlockSpec((BLK, D), lambda i: (i, 0)),
            pl.BlockSpec((D, 64), lambda i: (0, 0)),
            pl.BlockSpec((1, 64), lambda i: (0, 0)),
            pl.BlockSpec((64, 4), lambda i: (0, 0)),
            pl.BlockSpec((1, 4), lambda i: (0, 0)),
        ],
        out_specs=[
            pl.BlockSpec((1, 1, BLK), lambda i: (i, 0, 0)),
            pl.BlockSpec((BLK, 1), lambda i: (i, 0)),
            pl.BlockSpec((BLK, 1), lambda i: (i, 0)),
            pl.BlockSpec((BLK, 1), lambda i: (i, 0)),
            pl.BlockSpec((2, D), lambda i: (0, 0)),
        ],
        out_shape=[
            jax.ShapeDtypeStruct((NBLK, 1, BLK), jnp.int32),
            jax.ShapeDtypeStruct((S, 1), jnp.float32),
            jax.ShapeDtypeStruct((S, 1), jnp.float32),
            jax.ShapeDtypeStruct((S, 1), jnp.float32),
            jax.ShapeDtypeStruct((2, D), jnp.float32),
        ],
    )(hs2d, p['r_w1'], p['r_b1'].reshape(1, 64),
      p['r_w2'], p['r_b2'].reshape(1, 4))
    return cls3.reshape(S), s01, s11, s00, m10stats


def _cnn_kernel(prev_ref, cur_ref, next_ref, dww_ref, dwb_ref,
                pww_ref, pwb_ref, cg_ref, cb_ref, cnn_ref):
    i = pl.program_id(0)
    x = cur_ref[...]
    zero2 = jnp.zeros((2, D), jnp.float32)
    top = jnp.where(i > 0, prev_ref[BLK - 2:BLK, :], zero2)
    bot = jnp.where(i < NBLK - 1, next_ref[0:2, :], zero2)
    ext = jnp.concatenate([top, x, bot], axis=0)
    acc = dwb_ref[...]
    for j in range(KS):
        acc = acc + ext[j:j + BLK, :] * dww_ref[j:j + 1, :]
    conv = _gelu(acc)
    y = _bdot(conv, pww_ref[...]) + pwb_ref[...]
    cnn_ref[...] = _ln(y, cg_ref[...], cb_ref[...])


def _cnn(hs2d, p):
    dww = jnp.transpose(p['dw_w'][:, 0, :], (1, 0))
    return pl.pallas_call(
        _cnn_kernel,
        grid=(NBLK,),
        in_specs=[
            pl.BlockSpec((BLK, D), lambda i: (jnp.maximum(i - 1, 0), 0)),
            pl.BlockSpec((BLK, D), lambda i: (i, 0)),
            pl.BlockSpec((BLK, D), lambda i: (jnp.minimum(i + 1, NBLK - 1), 0)),
            pl.BlockSpec((KS, D), lambda i: (0, 0)),
            pl.BlockSpec((1, D), lambda i: (0, 0)),
            pl.BlockSpec((D, D), lambda i: (0, 0)),
            pl.BlockSpec((1, D), lambda i: (0, 0)),
            pl.BlockSpec((1, D), lambda i: (0, 0)),
            pl.BlockSpec((1, D), lambda i: (0, 0)),
        ],
        out_specs=pl.BlockSpec((BLK, D), lambda i: (i, 0)),
        out_shape=jax.ShapeDtypeStruct((S, D), jnp.float32),
    )(hs2d, hs2d, hs2d, dww, p['dw_b'].reshape(1, D),
      p['pw_w'].astype(jnp.bfloat16), p['pw_b'].reshape(1, D),
      p['cnn_g'].reshape(1, D), p['cnn_b'].reshape(1, D))


def _register_kernel(reg_ref, m10stats_ref, alpha_ref,
                     regw_ref, regb_ref, reglng_ref, reglnb_ref,
                     wrk_ref, brk_ref, wrv_ref, brv_ref,
                     upd_ref, kr_ref, vr_ref):
    cnt = jnp.maximum(m10stats_ref[1:2, :], 1.0)
    anc_mean = _dot(m10stats_ref[0:1, :], regw_ref[...]) / cnt + regb_ref[...]
    reg = reg_ref[...]
    a = jax.nn.sigmoid(alpha_ref[...])
    upd_ref[...] = _ln(reg + a * (anc_mean - reg), reglng_ref[...], reglnb_ref[...])
    kr_ref[...] = _dot(reg, wrk_ref[...]) + brk_ref[...]
    vr_ref[...] = _dot(reg, wrv_ref[...]) + brv_ref[...]


def _register(reg2d, m10stats, p):
    full = lambda shp: pl.BlockSpec(shp, lambda: (0,) * len(shp))
    return pl.pallas_call(
        _register_kernel,
        in_specs=[full((1, D)), full((2, D)), full((1, 1)),
                  full((D, D)), full((1, D)), full((1, D)), full((1, D)),
                  full((D, D)), full((1, D)), full((D, D)), full((1, D))],
        out_specs=[full((1, D)), full((1, D)), full((1, D))],
        out_shape=[jax.ShapeDtypeStruct((1, D), jnp.float32)] * 3,
    )(reg2d, m10stats, p['alpha'].reshape(1, 1),
      p['reg_w'], p['reg_b'].reshape(1, D),
      p['regln_g'].reshape(1, D), p['regln_b'].reshape(1, D),
      p['wrk'], p['brk'].reshape(1, D), p['wrv'], p['brv'].reshape(1, D))


def _compact(cls):
    mesh = plsc.ScalarSubcoreMesh(axis_name='core', num_cores=1)

    @pl.kernel(
        out_type=[jax.ShapeDtypeStruct((S,), jnp.int32),
                  jax.ShapeDtypeStruct((S,), jnp.int32),
                  jax.ShapeDtypeStruct((16,), jnp.int32)],
        mesh=mesh,
        scratch_types=[pltpu.SMEM((S,), jnp.int32),
                       pltpu.SMEM((S,), jnp.int32),
                       pltpu.SMEM((S,), jnp.int32),
                       pltpu.SMEM((16,), jnp.int32),
                       pltpu.SemaphoreType.DMA],
    )
    def body(cls_hbm, idxg_hbm, idxs_hbm, cnt_hbm,
             cls_s, idxg_s, idxs_s, cnt_s, sem):
        pltpu.async_copy(cls_hbm, cls_s, sem).wait()

        @pl.loop(0, 16)
        def _(i):
            cnt_s[i] = 0

        @pl.loop(0, S)
        def _(i):
            @pl.when(cls_s[i] == 3)
            def _():
                p = cnt_s[0]
                idxg_s[p] = i
                idxs_s[p] = i
                cnt_s[0] = p + 1

        n = cnt_s[0]
        hi = jnp.minimum(((n + _CHUNK - 1) // _CHUNK) * _CHUNK, S)

        @pl.loop(n, hi)
        def _(i):
            idxg_s[i] = 0
            idxs_s[i] = S

        c1 = pltpu.async_copy(idxg_s, idxg_hbm, sem)
        c2 = pltpu.async_copy(idxs_s, idxs_hbm, sem)
        c3 = pltpu.async_copy(cnt_s, cnt_hbm, sem)
        c1.wait()
        c2.wait()
        c3.wait()

    idxg, idxs, cnts = body(cls)
    return idxg, idxs, cnts


def _sc_vec_params():
    cp = pltpu.CompilerParams()
    if "needs_layout_passes" in pltpu.CompilerParams.__dataclass_fields__:
        cp = dataclasses.replace(cp, needs_layout_passes=False)
    return cp


def _sc_gather(table, idx, cnts):
    mesh = plsc.VectorSubcoreMesh(core_axis_name='c', subcore_axis_name='s')

    @pl.kernel(
        out_type=jax.ShapeDtypeStruct((S, D), jnp.float32),
        mesh=mesh,
        compiler_params=_sc_vec_params(),
        scratch_types=[pltpu.VMEM((_CHUNK,), jnp.int32),
                       pltpu.VMEM((_CHUNK, D), jnp.float32),
                       pltpu.VMEM((16,), jnp.int32),
                       pltpu.SemaphoreType.DMA],
    )
    def body(table_hbm, idx_hbm, cnt_hbm, out_hbm, idx_v, rows_v, cnt_v, sem):
        pltpu.sync_copy(cnt_hbm, cnt_v)
        n = jnp.max(cnt_v[...])
        wid = jax.lax.axis_index('s') * 2 + jax.lax.axis_index('c')
        for c in range(_CPS):
            j = wid + c * _NSUB

            @pl.when(j * _CHUNK < n)
            def _(j=j):
                base = j * _CHUNK
                pltpu.sync_copy(idx_hbm.at[pl.ds(base, _CHUNK)], idx_v)
                pltpu.async_copy(table_hbm.at[idx_v], rows_v, sem).wait()
                pltpu.sync_copy(rows_v, out_hbm.at[pl.ds(base, _CHUNK)])

    return body(table, idx, cnts)


def _sc_scatter(rows, idx, cnts):
    mesh = plsc.VectorSubcoreMesh(core_axis_name='c', subcore_axis_name='s')

    @pl.kernel(
        out_type=jax.ShapeDtypeStruct((S + _CHUNK, D), jnp.float32),
        mesh=mesh,
        compiler_params=_sc_vec_params(),
        scratch_types=[pltpu.VMEM((_CHUNK,), jnp.int32),
                       pltpu.VMEM((_CHUNK, D), jnp.float32),
                       pltpu.VMEM((16,), jnp.int32),
                       pltpu.SemaphoreType.DMA],
    )
    def body(rows_hbm, idx_hbm, cnt_hbm, out_hbm, idx_v, rows_v, cnt_v, sem):
        pltpu.sync_copy(cnt_hbm, cnt_v)
        n = jnp.max(cnt_v[...])
        wid = jax.lax.axis_index('s') * 2 + jax.lax.axis_index('c')
        for c in range(_CPS):
            j = wid + c * _NSUB

            @pl.when(j * _CHUNK < n)
            def _(j=j):
                base = j * _CHUNK
                pltpu.sync_copy(idx_hbm.at[pl.ds(base, _CHUNK)], idx_v)
                pltpu.sync_copy(rows_hbm.at[pl.ds(base, _CHUNK)], rows_v)
                pltpu.async_copy(rows_v, out_hbm.at[idx_v], sem).wait()

    return body(rows, idx, cnts)


def _qkv_kernel(n_ref, x_ref, wq_ref, bq_ref, wk_ref, bk_ref, wv_ref, bv_ref,
                q_ref, k_ref, v_ref):
    i = pl.program_id(0)

    @pl.when(i * BLK < n_ref[0])
    def _():
        ridx = i * BLK + jax.lax.broadcasted_iota(jnp.int32, (BLK, 1), 0)
        x = jnp.where(ridx < n_ref[0], x_ref[...], 0.0).astype(jnp.bfloat16)
        q_ref[...] = _dot(x, wq_ref[...]) + bq_ref[...]
        k_ref[...] = _dot(x, wk_ref[...]) + bk_ref[...]
        v_ref[...] = _dot(x, wv_ref[...]) + bv_ref[...]


def _qkv(x11, n11, p):
    grid_spec = pltpu.PrefetchScalarGridSpec(
        num_scalar_prefetch=1,
        grid=(NBLK,),
        in_specs=[
            pl.BlockSpec((BLK, D), lambda i, n: (i, 0)),
            pl.BlockSpec((D, D), lambda i, n: (0, 0)),
            pl.BlockSpec((1, D), lambda i, n: (0, 0)),
            pl.BlockSpec((D, D), lambda i, n: (0, 0)),
            pl.BlockSpec((1, D), lambda i, n: (0, 0)),
            pl.BlockSpec((D, D), lambda i, n: (0, 0)),
            pl.BlockSpec((1, D), lambda i, n: (0, 0)),
        ],
        out_specs=[pl.BlockSpec((BLK, D), lambda i, n: (i, 0))] * 3,
    )
    return pl.pallas_call(
        _qkv_kernel,
        grid_spec=grid_spec,
        out_shape=[jax.ShapeDtypeStruct((S, D), jnp.float32)] * 3,
    )(n11, x11, p['wq'].astype(jnp.bfloat16), p['bq'].reshape(1, D),
      p['wk'].astype(jnp.bfloat16), p['bk'].reshape(1, D),
      p['wv'].astype(jnp.bfloat16), p['bv'].reshape(1, D))


def _attn_kernel(n_ref, q_ref, k_ref, v_ref, kr_ref, vr_ref, ao_ref):
    qi = pl.program_id(1)
    n = n_ref[0]
    nblk = (n + BLK - 1) // BLK

    @pl.when(qi * BLK < n)
    def _():
        for t in range(1, NBLK + 1):
            @pl.when(nblk == t)
            def _(t=t):
                outs = []
                for off in (0, DH):
                    q = q_ref[:, off:off + DH]
                    qb = q.astype(jnp.bfloat16)
                    sreg = (q * kr_ref[:, off:off + DH]).sum(
                        axis=-1, keepdims=True) * INV_SCALE
                    kt = k_ref[0:t * BLK, off:off + DH]
                    s = _dot(qb, kt.astype(jnp.bfloat16).T) * INV_SCALE
                    kidx = jax.lax.broadcasted_iota(
                        jnp.int32, (BLK, t * BLK), 1)
                    s = jnp.where(kidx < n, s, -1e9)
                    m = jnp.maximum(s.max(axis=-1, keepdims=True), sreg)
                    w = jnp.exp(s - m)
                    wr = jnp.exp(sreg - m)
                    den = w.sum(axis=-1, keepdims=True) + wr
                    vt = v_ref[0:t * BLK, off:off + DH]
                    o = _dot(w.astype(jnp.bfloat16), vt.astype(jnp.bfloat16))
                    outs.append((o + wr * vr_ref[:, off:off + DH]) / den)
                ao_ref[...] = jnp.concatenate(outs, axis=1)


def _attention(q, k, v, kr, vr, n11):
    grid_spec = pltpu.PrefetchScalarGridSpec(
        num_scalar_prefetch=1,
        grid=(H // 2, NBLK),
        in_specs=[
            pl.BlockSpec((BLK, 2 * DH), lambda h, qi, n: (qi, h)),
            pl.BlockSpec((S, 2 * DH), lambda h, qi, n: (0, h)),
            pl.BlockSpec((S, 2 * DH), lambda h, qi, n: (0, h)),
            pl.BlockSpec((1, 2 * DH), lambda h, qi, n: (0, h)),
            pl.BlockSpec((1, 2 * DH), lambda h, qi, n: (0, h)),
        ],
        out_specs=pl.BlockSpec((BLK, 2 * DH), lambda h, qi, n: (qi, h)),
    )
    return pl.pallas_call(
        _attn_kernel,
        grid_spec=grid_spec,
        out_shape=jax.ShapeDtypeStruct((S, D), jnp.float32),
    )(n11, q, k, v, kr, vr)


def _oproj_kernel(n_ref, x_ref, w_ref, b_ref, o_ref):
    @pl.when(pl.program_id(0) * BLK < n_ref[0])
    def _():
        o_ref[...] = _bdot(x_ref[...], w_ref[...]) + b_ref[...]


def _oproj(x, n, p):
    grid_spec = pltpu.PrefetchScalarGridSpec(
        num_scalar_prefetch=1,
        grid=(NBLK,),
        in_specs=[
            pl.BlockSpec((BLK, D), lambda i, n: (i, 0)),
            pl.BlockSpec((D, D), lambda i, n: (0, 0)),
            pl.BlockSpec((1, D), lambda i, n: (0, 0)),
        ],
        out_specs=pl.BlockSpec((BLK, D), lambda i, n: (i, 0)),
    )
    return pl.pallas_call(
        _oproj_kernel,
        grid_spec=grid_spec,
        out_shape=jax.ShapeDtypeStruct((S, D), jnp.float32),
    )(n, x, p['wo'].astype(jnp.bfloat16), p['bo'].reshape(1, D))


def _ffn_kernel(hs_ref, cnn_ref, attn_ref, s01_ref, s11_ref, s00_ref,
                w1_ref, b1_ref, w2_ref, b2_ref,
                g1_ref, lb1_ref, g2_ref, lb2_ref, out_ref):
    hs = hs_ref[...]
    combined = jnp.where(s00_ref[...] > 0.5, 0.0, hs)
    combined = jnp.where(s01_ref[...] > 0.5, cnn_ref[...], combined)
    combined = jnp.where(s11_ref[...] > 0.5, attn_ref[...], combined)
    x1 = _ln(hs + combined, g1_ref[...], lb1_ref[...])
    t = _gelu(_bdot(x1, w1_ref[...]) + b1_ref[...])
    f = _bdot(t, w2_ref[...]) + b2_ref[...]
    out_ref[...] = _ln(x1 + f, g2_ref[...], lb2_ref[...])


def _ffn(hs2d, cnn, attn, s01, s11, s00, p):
    return pl.pallas_call(
        _ffn_kernel,
        grid=(NBLK,),
        in_specs=[
            pl.BlockSpec((BLK, D), lambda i: (i, 0)),
            pl.BlockSpec((BLK, D), lambda i: (i, 0)),
            pl.BlockSpec((BLK, D), lambda i: (i, 0)),
            pl.BlockSpec((BLK, 1), lambda i: (i, 0)),
            pl.BlockSpec((BLK, 1), lambda i: (i, 0)),
            pl.BlockSpec((BLK, 1), lambda i: (i, 0)),
            pl.BlockSpec((D, FF), lambda i: (0, 0)),
            pl.BlockSpec((1, FF), lambda i: (0, 0)),
            pl.BlockSpec((FF, D), lambda i: (0, 0)),
            pl.BlockSpec((1, D), lambda i: (0, 0)),
            pl.BlockSpec((1, D), lambda i: (0, 0)),
            pl.BlockSpec((1, D), lambda i: (0, 0)),
            pl.BlockSpec((1, D), lambda i: (0, 0)),
            pl.BlockSpec((1, D), lambda i: (0, 0)),
        ],
        out_specs=pl.BlockSpec((BLK, D), lambda i: (i, 0)),
        out_shape=jax.ShapeDtypeStruct((S, D), jnp.float32),
    )(hs2d, cnn, attn, s01, s11, s00,
      p['ffn_w1'].astype(jnp.bfloat16), p['ffn_b1'].reshape(1, FF),
      p['ffn_w2'].astype(jnp.bfloat16), p['ffn_b2'].reshape(1, D),
      p['ln1_g'].reshape(1, D), p['ln1_b'].reshape(1, D),
      p['ln2_g'].reshape(1, D), p['ln2_b'].reshape(1, D))


def kernel(hidden_states, register_state, params):
    p = params
    hs2d = hidden_states.reshape(S, D)
    reg2d = register_state.reshape(1, D)

    cls, s01, s11, s00, m10stats = _router(hs2d, p)
    idxg, idxs, cnts = _compact(cls)
    cnn = _cnn(hs2d, p)
    upd, kr, vr = _register(reg2d, m10stats, p)
    n11 = cnts[0:1]

    x11 = _sc_gather(hs2d, idxg, cnts)
    q, k, v = _qkv(x11, n11, p)
    ao = _attention(q, k, v, kr, vr, n11)
    attn_cmp = _oproj(ao, n11, p)
    attn_pos = _sc_scatter(attn_cmp, idxs, cnts)

    out = _ffn(hs2d, cnn, attn_pos, s01, s11, s00, p)
    return out.reshape(1, S, D), upd.reshape(1, D)
